# Initial kernel scaffold; baseline (speedup 1.0000x reference)
import functools

import jax
import jax.numpy as jnp
from jax import lax
from jax.experimental import pallas as pl
from jax.experimental.pallas import tpu as pltpu
from jax.experimental.pallas import tpu_sc as plsc

N = 10000
NP = 10240
E = 320000
NSC = 2
NSUB = 16
EB = 80

DIMS_P = [128, 224, 160, 128, 64, 32]
ROWS_PER_SUB = NP // NSUB

_mesh = functools.partial(
    plsc.VectorSubcoreMesh,
    core_axis_name="c", subcore_axis_name="s",
    num_cores=NSC, num_subcores=NSUB,
)


def _sc_degree_body(dst, ones, zeros, deg0, deg1, acc, dib, obuf, sem):
    s = lax.axis_index("s")
    c = lax.axis_index("c")
    r0 = s * ROWS_PER_SUB
    pltpu.sync_copy(zeros, acc.at[pl.ds(r0, ROWS_PER_SUB)])
    pltpu.sync_copy(ones, obuf)
    plsc.subcore_barrier()

    w = s * NSC + c
    base = w * (E // (NSC * NSUB))

    @pl.loop(0, (E // (NSC * NSUB)) // EB)
    def _batches(i):
        b = pl.multiple_of(base + i * EB, 8)
        pltpu.async_copy(dst.at[pl.ds(b, EB)], dib, sem).wait()
        pltpu.sync_copy(obuf, acc.at[dib], add=True)

    plsc.subcore_barrier()

    @pl.when(c == 0)
    def _():
        pltpu.sync_copy(acc.at[pl.ds(r0, ROWS_PER_SUB)],
                        deg0.at[pl.ds(r0, ROWS_PER_SUB)])

    @pl.when(c == 1)
    def _():
        pltpu.sync_copy(acc.at[pl.ds(r0, ROWS_PER_SUB)],
                        deg1.at[pl.ds(r0, ROWS_PER_SUB)])


@functools.lru_cache(maxsize=None)
def _sc_degree_kernel():
    return pl.kernel(
        _sc_degree_body,
        out_type=(jax.ShapeDtypeStruct((NP, 1), jnp.float32),
                  jax.ShapeDtypeStruct((NP, 1), jnp.float32)),
        mesh=_mesh(),
        scratch_types=[
            pltpu.VMEM_SHARED((NP, 1), jnp.float32),
            pltpu.VMEM((EB,), jnp.int32),
            pltpu.VMEM((EB, 1), jnp.float32),
            pltpu.SemaphoreType.DMA,
        ],
    )


def _sc_agg_body(src, dst, z0, z1, zeros, out0, out1,
                 acc, sib, dib, ebuf, sem, semg):
    s = lax.axis_index("s")
    c = lax.axis_index("c")
    r0 = s * ROWS_PER_SUB
    pltpu.sync_copy(zeros, acc.at[pl.ds(r0, ROWS_PER_SUB)])
    plsc.subcore_barrier()

    e_per_sub = E // NSUB
    base = s * e_per_sub

    @pl.loop(0, e_per_sub // EB)
    def _batches(i):
        b = pl.multiple_of(base + i * EB, 8)
        pltpu.async_copy(src.at[pl.ds(b, EB)], sib, sem).wait()
        pltpu.async_copy(dst.at[pl.ds(b, EB)], dib, sem).wait()

        @pl.when(c == 0)
        def _():
            pltpu.async_copy(z0.at[sib], ebuf, semg).wait()

        @pl.when(c == 1)
        def _():
            pltpu.async_copy(z1.at[sib], ebuf, semg).wait()

        pltpu.sync_copy(ebuf, acc.at[dib], add=True)

    plsc.subcore_barrier()

    @pl.when(c == 0)
    def _():
        pltpu.sync_copy(acc.at[pl.ds(r0, ROWS_PER_SUB)],
                        out0.at[pl.ds(r0, ROWS_PER_SUB)])

    @pl.when(c == 1)
    def _():
        pltpu.sync_copy(acc.at[pl.ds(r0, ROWS_PER_SUB)],
                        out1.at[pl.ds(r0, ROWS_PER_SUB)])


@functools.lru_cache(maxsize=None)
def _sc_agg_kernel(d2):
    return pl.kernel(
        _sc_agg_body,
        out_type=(jax.ShapeDtypeStruct((NP, d2), jnp.float32),
                  jax.ShapeDtypeStruct((NP, d2), jnp.float32)),
        mesh=_mesh(),
        scratch_types=[
            pltpu.VMEM_SHARED((NP, d2), jnp.float32),
            pltpu.VMEM((EB,), jnp.int32),
            pltpu.VMEM((EB,), jnp.int32),
            pltpu.VMEM((EB, d2), jnp.float32),
            pltpu.SemaphoreType.DMA,
            pltpu.SemaphoreType.DMA,
        ],
    )


_BLK = 256
_GRID = NP // _BLK


def _row_spec(d):
    return pl.BlockSpec((_BLK, d), lambda i: (i, 0))


def _vec_spec(d):
    return pl.BlockSpec((1, d), lambda i: (0, 0))


def _w_spec(din, dout):
    return pl.BlockSpec((din, dout), lambda i: (0, 0))


def _tc_first_body(x, w, deg0, deg1, z0, z1, dis):
    d = deg0[...] + deg1[...]
    di = lax.rsqrt(jnp.maximum(d, 1.0))
    z = di * jnp.dot(x[...], w[...], preferred_element_type=jnp.float32)
    h = z.shape[1] // 2
    z0[...] = z[:, :h]
    z1[...] = z[:, h:]
    dis[...] = di


@functools.lru_cache(maxsize=None)
def _tc_first_kernel(din, dout):
    h = dout // 2
    return pl.pallas_call(
        _tc_first_body,
        grid=(_GRID,),
        in_specs=[_row_spec(din), _w_spec(din, dout),
                  _row_spec(1), _row_spec(1)],
        out_specs=(_row_spec(h), _row_spec(h), _row_spec(1)),
        out_shape=(jax.ShapeDtypeStruct((NP, h), jnp.float32),
                   jax.ShapeDtypeStruct((NP, h), jnp.float32),
                   jax.ShapeDtypeStruct((NP, 1), jnp.float32)),
    )


def _tc_mid_body(eps, s0, s1, z0, z1, dis, b, g, be, rm, rv, w, o0, o1):
    t = jnp.concatenate([s0[...] + z0[...], s1[...] + z1[...]], axis=1)
    pre = dis[...] * t + b[...]
    hrelu = jnp.maximum(pre, 0.0)
    a = g[...] * lax.rsqrt(rv[...] + eps)
    hbn = (hrelu - rm[...]) * a + be[...]
    z = dis[...] * jnp.dot(hbn, w[...], preferred_element_type=jnp.float32)
    h = z.shape[1] // 2
    o0[...] = z[:, :h]
    o1[...] = z[:, h:]


@functools.lru_cache(maxsize=None)
def _tc_mid_kernel(din, dout):
    hin, hout = din // 2, dout // 2
    return pl.pallas_call(
        functools.partial(_tc_mid_body, 1e-5),
        grid=(_GRID,),
        in_specs=[_row_spec(hin), _row_spec(hin), _row_spec(hin),
                  _row_spec(hin), _row_spec(1),
                  _vec_spec(din), _vec_spec(din), _vec_spec(din),
                  _vec_spec(din), _vec_spec(din), _w_spec(din, dout)],
        out_specs=(_row_spec(hout), _row_spec(hout)),
        out_shape=(jax.ShapeDtypeStruct((NP, hout), jnp.float32),
                   jax.ShapeDtypeStruct((NP, hout), jnp.float32)),
    )


def _tc_last_body(ncls, s0, s1, z0, z1, dis, b, o):
    t = jnp.concatenate([s0[...] + z0[...], s1[...] + z1[...]], axis=1)
    t = dis[...] * t + b[...]
    col = lax.broadcasted_iota(jnp.int32, t.shape, 1)
    mask = col < ncls
    neg = jnp.float32(-3.4e38)
    m = jnp.max(jnp.where(mask, t, neg), axis=1, keepdims=True)
    ex = jnp.where(mask, jnp.exp(t - m), 0.0)
    lse = jnp.log(jnp.sum(ex, axis=1, keepdims=True))
    o[...] = t - m - lse


@functools.lru_cache(maxsize=None)
def _tc_last_kernel(din, ncls):
    hin = din // 2
    return pl.pallas_call(
        functools.partial(_tc_last_body, ncls),
        grid=(_GRID,),
        in_specs=[_row_spec(hin), _row_spec(hin), _row_spec(hin),
                  _row_spec(hin), _row_spec(1), _vec_spec(din)],
        out_specs=_row_spec(din),
        out_shape=jax.ShapeDtypeStruct((NP, din), jnp.float32),
    )


def _pad2(a, r, c):
    return jnp.pad(a, ((0, r - a.shape[0]), (0, c - a.shape[1])))


def _padv(a, c, value=0.0):
    return jnp.pad(a, (0, c - a.shape[0]), constant_values=value)[None, :]


def kernel(x, edge_index, W1, b1, W2, b2, W3, b3, W4, b4, W5, b5,
           g1, be1, rm1, rv1, g2, be2, rm2, rv2, g3, be3, rm3, rv3,
           g4, be4, rm4, rv4):
    src = edge_index[0]
    dst = edge_index[1]

    Ws = [W1, W2, W3, W4, W5]
    bs = [b1, b2, b3, b4, b5]
    gs = [g1, g2, g3, g4]
    bes = [be1, be2, be3, be4]
    rms = [rm1, rm2, rm3, rm4]
    rvs = [rv1, rv2, rv3, rv4]

    xp = _pad2(x, NP, DIMS_P[0])
    wp = [_pad2(Ws[i], DIMS_P[i], DIMS_P[i + 1]) for i in range(5)]
    bp = [_padv(bs[i], DIMS_P[i + 1]) for i in range(5)]
    gp = [_padv(gs[i], DIMS_P[i + 1]) for i in range(4)]
    bep = [_padv(bes[i], DIMS_P[i + 1]) for i in range(4)]
    rmp = [_padv(rms[i], DIMS_P[i + 1]) for i in range(4)]
    rvp = [_padv(rvs[i], DIMS_P[i + 1], 1.0) for i in range(4)]

    zeros1 = jnp.zeros((ROWS_PER_SUB, 1), jnp.float32)
    ones1 = jnp.ones((EB, 1), jnp.float32)

    deg0, deg1 = _sc_degree_kernel()(dst, ones1, zeros1)
    z0, z1, dis = _tc_first_kernel(DIMS_P[0], DIMS_P[1])(xp, wp[0], deg0, deg1)

    for l in range(1, 5):
        d2 = DIMS_P[l] // 2
        zer = jnp.zeros((ROWS_PER_SUB, d2), jnp.float32)
        s0, s1 = _sc_agg_kernel(d2)(src, dst, z0, z1, zer)
        z0, z1 = _tc_mid_kernel(DIMS_P[l], DIMS_P[l + 1])(
            s0, s1, z0, z1, dis, bp[l - 1], gp[l - 1], bep[l - 1],
            rmp[l - 1], rvp[l - 1], wp[l])

    d2 = DIMS_P[5] // 2
    zer = jnp.zeros((ROWS_PER_SUB, d2), jnp.float32)
    s0, s1 = _sc_agg_kernel(d2)(src, dst, z0, z1, zer)
    out = _tc_last_kernel(DIMS_P[5], 17)(s0, s1, z0, z1, dis, bp[4])
    return out[:N, :17]


# speedup vs baseline: 6.4930x; 6.4930x over previous
import functools

import jax
import jax.numpy as jnp
from jax import lax
from jax.experimental import pallas as pl
from jax.experimental.pallas import tpu as pltpu
from jax.experimental.pallas import tpu_sc as plsc

N = 10000
NP = 10240
E = 320000
NSC = 2
NSUB = 16
EB = 80

DIMS_P = [128, 224, 160, 128, 64, 32]
ROWS_PER_SUB = NP // NSUB
DW = 16

_mesh = functools.partial(
    plsc.VectorSubcoreMesh,
    core_axis_name="c", subcore_axis_name="s",
    num_cores=NSC, num_subcores=NSUB,
)

_SC_PARAMS = pltpu.CompilerParams(use_tc_tiling_on_sc=False)


def _sc_degree_body(dst, ones, zeros, deg0, deg1, acc, dib, obuf, sem):
    s = lax.axis_index("s")
    c = lax.axis_index("c")
    r0 = s * ROWS_PER_SUB
    pltpu.sync_copy(zeros, acc.at[pl.ds(r0, ROWS_PER_SUB)])
    pltpu.sync_copy(ones, obuf)
    plsc.subcore_barrier()

    w = s * NSC + c
    base = w * (E // (NSC * NSUB))

    @pl.loop(0, (E // (NSC * NSUB)) // EB)
    def _batches(i):
        b = pl.multiple_of(base + i * EB, 8)
        pltpu.async_copy(dst.at[pl.ds(b, EB)], dib, sem).wait()
        pltpu.sync_copy(obuf, acc.at[dib], add=True)

    plsc.subcore_barrier()

    @pl.when(c == 0)
    def _():
        pltpu.sync_copy(acc.at[pl.ds(r0, ROWS_PER_SUB)],
                        deg0.at[pl.ds(r0, ROWS_PER_SUB)])

    @pl.when(c == 1)
    def _():
        pltpu.sync_copy(acc.at[pl.ds(r0, ROWS_PER_SUB)],
                        deg1.at[pl.ds(r0, ROWS_PER_SUB)])


@functools.lru_cache(maxsize=None)
def _sc_degree_kernel():
    return pl.kernel(
        _sc_degree_body,
        out_type=(jax.ShapeDtypeStruct((NP, DW), jnp.float32),
                  jax.ShapeDtypeStruct((NP, DW), jnp.float32)),
        mesh=_mesh(),
        compiler_params=_SC_PARAMS,
        scratch_types=[
            pltpu.VMEM_SHARED((NP, DW), jnp.float32),
            pltpu.VMEM((EB,), jnp.int32),
            pltpu.VMEM((EB, DW), jnp.float32),
            pltpu.SemaphoreType.DMA,
        ],
    )


def _sc_agg_body(src, dst, z0, z1, zeros, out0, out1,
                 acc, sib, dib, ebuf, sem, semg):
    s = lax.axis_index("s")
    c = lax.axis_index("c")
    r0 = s * ROWS_PER_SUB
    pltpu.sync_copy(zeros, acc.at[pl.ds(r0, ROWS_PER_SUB)])
    plsc.subcore_barrier()

    e_per_sub = E // NSUB
    base = s * e_per_sub

    @pl.loop(0, e_per_sub // EB)
    def _batches(i):
        b = pl.multiple_of(base + i * EB, 8)
        pltpu.async_copy(src.at[pl.ds(b, EB)], sib, sem).wait()
        pltpu.async_copy(dst.at[pl.ds(b, EB)], dib, sem).wait()

        @pl.when(c == 0)
        def _():
            pltpu.async_copy(z0.at[sib], ebuf, semg).wait()

        @pl.when(c == 1)
        def _():
            pltpu.async_copy(z1.at[sib], ebuf, semg).wait()

        pltpu.sync_copy(ebuf, acc.at[dib], add=True)

    plsc.subcore_barrier()

    @pl.when(c == 0)
    def _():
        pltpu.sync_copy(acc.at[pl.ds(r0, ROWS_PER_SUB)],
                        out0.at[pl.ds(r0, ROWS_PER_SUB)])

    @pl.when(c == 1)
    def _():
        pltpu.sync_copy(acc.at[pl.ds(r0, ROWS_PER_SUB)],
                        out1.at[pl.ds(r0, ROWS_PER_SUB)])


@functools.lru_cache(maxsize=None)
def _sc_agg_kernel(d2):
    return pl.kernel(
        _sc_agg_body,
        out_type=(jax.ShapeDtypeStruct((NP, d2), jnp.float32),
                  jax.ShapeDtypeStruct((NP, d2), jnp.float32)),
        mesh=_mesh(),
        compiler_params=_SC_PARAMS,
        scratch_types=[
            pltpu.VMEM_SHARED((NP, d2), jnp.float32),
            pltpu.VMEM((EB,), jnp.int32),
            pltpu.VMEM((EB,), jnp.int32),
            pltpu.VMEM((EB, d2), jnp.float32),
            pltpu.SemaphoreType.DMA,
            pltpu.SemaphoreType.DMA,
        ],
    )


_BLK = 256
_GRID = NP // _BLK


def _row_spec(d):
    return pl.BlockSpec((_BLK, d), lambda i: (i, 0))


def _vec_spec(d):
    return pl.BlockSpec((1, d), lambda i: (0, 0))


def _w_spec(din, dout):
    return pl.BlockSpec((din, dout), lambda i: (0, 0))


def _tc_first_body(x, w, deg0, deg1, z0, z1, dis):
    d = deg0[...][:, :1] + deg1[...][:, :1]
    di = lax.rsqrt(jnp.maximum(d, 1.0))
    z = di * jnp.dot(x[...], w[...], preferred_element_type=jnp.float32)
    h = z.shape[1] // 2
    z0[...] = z[:, :h]
    z1[...] = z[:, h:]
    dis[...] = di


@functools.lru_cache(maxsize=None)
def _tc_first_kernel(din, dout):
    h = dout // 2
    return pl.pallas_call(
        _tc_first_body,
        grid=(_GRID,),
        in_specs=[_row_spec(din), _w_spec(din, dout),
                  _row_spec(DW), _row_spec(DW)],
        out_specs=(_row_spec(h), _row_spec(h), _row_spec(1)),
        out_shape=(jax.ShapeDtypeStruct((NP, h), jnp.float32),
                   jax.ShapeDtypeStruct((NP, h), jnp.float32),
                   jax.ShapeDtypeStruct((NP, 1), jnp.float32)),
    )


def _tc_mid_body(eps, s0, s1, z0, z1, dis, b, g, be, rm, rv, w, o0, o1):
    t = jnp.concatenate([s0[...] + z0[...], s1[...] + z1[...]], axis=1)
    pre = dis[...] * t + b[...]
    hrelu = jnp.maximum(pre, 0.0)
    a = g[...] * lax.rsqrt(rv[...] + eps)
    hbn = (hrelu - rm[...]) * a + be[...]
    z = dis[...] * jnp.dot(hbn, w[...], preferred_element_type=jnp.float32)
    h = z.shape[1] // 2
    o0[...] = z[:, :h]
    o1[...] = z[:, h:]


@functools.lru_cache(maxsize=None)
def _tc_mid_kernel(din, dout):
    hin, hout = din // 2, dout // 2
    return pl.pallas_call(
        functools.partial(_tc_mid_body, 1e-5),
        grid=(_GRID,),
        in_specs=[_row_spec(hin), _row_spec(hin), _row_spec(hin),
                  _row_spec(hin), _row_spec(1),
                  _vec_spec(din), _vec_spec(din), _vec_spec(din),
                  _vec_spec(din), _vec_spec(din), _w_spec(din, dout)],
        out_specs=(_row_spec(hout), _row_spec(hout)),
        out_shape=(jax.ShapeDtypeStruct((NP, hout), jnp.float32),
                   jax.ShapeDtypeStruct((NP, hout), jnp.float32)),
    )


def _tc_last_body(ncls, s0, s1, z0, z1, dis, b, o):
    t = jnp.concatenate([s0[...] + z0[...], s1[...] + z1[...]], axis=1)
    t = dis[...] * t + b[...]
    col = lax.broadcasted_iota(jnp.int32, t.shape, 1)
    mask = col < ncls
    neg = jnp.float32(-3.4e38)
    m = jnp.max(jnp.where(mask, t, neg), axis=1, keepdims=True)
    ex = jnp.where(mask, jnp.exp(t - m), 0.0)
    lse = jnp.log(jnp.sum(ex, axis=1, keepdims=True))
    o[...] = t - m - lse


@functools.lru_cache(maxsize=None)
def _tc_last_kernel(din, ncls):
    hin = din // 2
    return pl.pallas_call(
        functools.partial(_tc_last_body, ncls),
        grid=(_GRID,),
        in_specs=[_row_spec(hin), _row_spec(hin), _row_spec(hin),
                  _row_spec(hin), _row_spec(1), _vec_spec(din)],
        out_specs=_row_spec(din),
        out_shape=jax.ShapeDtypeStruct((NP, din), jnp.float32),
    )


def _pad2(a, r, c):
    return jnp.pad(a, ((0, r - a.shape[0]), (0, c - a.shape[1])))


def _padv(a, c, value=0.0):
    return jnp.pad(a, (0, c - a.shape[0]), constant_values=value)[None, :]


def kernel(x, edge_index, W1, b1, W2, b2, W3, b3, W4, b4, W5, b5,
           g1, be1, rm1, rv1, g2, be2, rm2, rv2, g3, be3, rm3, rv3,
           g4, be4, rm4, rv4):
    src = edge_index[0]
    dst = edge_index[1]

    Ws = [W1, W2, W3, W4, W5]
    bs = [b1, b2, b3, b4, b5]
    gs = [g1, g2, g3, g4]
    bes = [be1, be2, be3, be4]
    rms = [rm1, rm2, rm3, rm4]
    rvs = [rv1, rv2, rv3, rv4]

    xp = _pad2(x, NP, DIMS_P[0])
    wp = [_pad2(Ws[i], DIMS_P[i], DIMS_P[i + 1]) for i in range(5)]
    bp = [_padv(bs[i], DIMS_P[i + 1]) for i in range(5)]
    gp = [_padv(gs[i], DIMS_P[i + 1]) for i in range(4)]
    bep = [_padv(bes[i], DIMS_P[i + 1]) for i in range(4)]
    rmp = [_padv(rms[i], DIMS_P[i + 1]) for i in range(4)]
    rvp = [_padv(rvs[i], DIMS_P[i + 1], 1.0) for i in range(4)]

    zeros1 = jnp.zeros((ROWS_PER_SUB, DW), jnp.float32)
    ones1 = jnp.ones((EB, DW), jnp.float32)

    deg0, deg1 = _sc_degree_kernel()(dst, ones1, zeros1)
    z0, z1, dis = _tc_first_kernel(DIMS_P[0], DIMS_P[1])(xp, wp[0], deg0, deg1)

    for l in range(1, 5):
        d2 = DIMS_P[l] // 2
        zer = jnp.zeros((ROWS_PER_SUB, d2), jnp.float32)
        s0, s1 = _sc_agg_kernel(d2)(src, dst, z0, z1, zer)
        z0, z1 = _tc_mid_kernel(DIMS_P[l], DIMS_P[l + 1])(
            s0, s1, z0, z1, dis, bp[l - 1], gp[l - 1], bep[l - 1],
            rmp[l - 1], rvp[l - 1], wp[l])

    d2 = DIMS_P[5] // 2
    zer = jnp.zeros((ROWS_PER_SUB, d2), jnp.float32)
    s0, s1 = _sc_agg_kernel(d2)(src, dst, z0, z1, zer)
    out = _tc_last_kernel(DIMS_P[5], 17)(s0, s1, z0, z1, dis, bp[4])
    return out[:N, :17]


# speedup vs baseline: 20.4036x; 3.1424x over previous
import functools

import jax
import jax.numpy as jnp
from jax import lax
from jax.experimental import pallas as pl
from jax.experimental.pallas import tpu as pltpu
from jax.experimental.pallas import tpu_sc as plsc

N = 10000
NP = 10240
E = 320000
NSC = 2
NSUB = 16
EB = 80

DIMS_P = [128, 224, 160, 128, 64, 32]
ROWS_PER_SUB = NP // NSUB
DW = 16
NB = (E // NSUB) // EB

_mesh = functools.partial(
    plsc.VectorSubcoreMesh,
    core_axis_name="c", subcore_axis_name="s",
    num_cores=NSC, num_subcores=NSUB,
)

_SC_PARAMS = pltpu.CompilerParams(use_tc_tiling_on_sc=False)


def _sc_degree_body(dst, ones, zeros, deg0, deg1, acc, dib, obuf, sem):
    s = lax.axis_index("s")
    c = lax.axis_index("c")
    r0 = s * ROWS_PER_SUB
    pltpu.sync_copy(zeros, acc.at[pl.ds(r0, ROWS_PER_SUB)])
    pltpu.sync_copy(ones, obuf)
    plsc.subcore_barrier()

    w = s * NSC + c
    base = w * (E // (NSC * NSUB))

    @pl.loop(0, (E // (NSC * NSUB)) // EB)
    def _batches(i):
        b = pl.multiple_of(base + i * EB, 8)
        pltpu.async_copy(dst.at[pl.ds(b, EB)], dib, sem).wait()
        pltpu.sync_copy(obuf, acc.at[dib], add=True)

    plsc.subcore_barrier()

    @pl.when(c == 0)
    def _():
        pltpu.sync_copy(acc.at[pl.ds(r0, ROWS_PER_SUB)],
                        deg0.at[pl.ds(r0, ROWS_PER_SUB)])

    @pl.when(c == 1)
    def _():
        pltpu.sync_copy(acc.at[pl.ds(r0, ROWS_PER_SUB)],
                        deg1.at[pl.ds(r0, ROWS_PER_SUB)])


@functools.lru_cache(maxsize=None)
def _sc_degree_kernel():
    return pl.kernel(
        _sc_degree_body,
        out_type=(jax.ShapeDtypeStruct((NP, DW), jnp.float32),
                  jax.ShapeDtypeStruct((NP, DW), jnp.float32)),
        mesh=_mesh(),
        compiler_params=_SC_PARAMS,
        scratch_types=[
            pltpu.VMEM_SHARED((NP, DW), jnp.float32),
            pltpu.VMEM((EB,), jnp.int32),
            pltpu.VMEM((EB, DW), jnp.float32),
            pltpu.SemaphoreType.DMA,
        ],
    )


def _sc_agg_body(nbuf, src3, dst3, z0, z1, zeros, out0, out1,
                 acc, sidx, didx, *bufs_and_sems):
    s = lax.axis_index("s")
    c = lax.axis_index("c")
    r0 = s * ROWS_PER_SUB
    ebufs = bufs_and_sems[:nbuf]
    gsems = bufs_and_sems[nbuf:2 * nbuf]
    ssems = bufs_and_sems[2 * nbuf:]

    pltpu.sync_copy(src3.at[s], sidx)
    pltpu.sync_copy(dst3.at[s], didx)
    pltpu.sync_copy(zeros, acc.at[pl.ds(r0, ROWS_PER_SUB)])

    def gissue(i, b):
        @pl.when(c == 0)
        def _():
            pltpu.async_copy(z0.at[sidx.at[i]], ebufs[b], gsems[b])

        @pl.when(c == 1)
        def _():
            pltpu.async_copy(z1.at[sidx.at[i]], ebufs[b], gsems[b])

    def gwait(i, b):
        @pl.when(c == 0)
        def _():
            pltpu.make_async_copy(z0.at[sidx.at[i]], ebufs[b], gsems[b]).wait()

        @pl.when(c == 1)
        def _():
            pltpu.make_async_copy(z1.at[sidx.at[i]], ebufs[b], gsems[b]).wait()

    def sissue(i, b):
        pltpu.async_copy(ebufs[b], acc.at[didx.at[i]], ssems[b], add=True)

    def swait(i, b):
        pltpu.make_async_copy(ebufs[b], acc.at[didx.at[i]], ssems[b]).wait()

    for b in range(nbuf):
        gissue(b, b)
    plsc.subcore_barrier()

    @pl.loop(0, NB // nbuf)
    def _segments(j):
        i0 = j * nbuf
        for b in range(nbuf):
            i = i0 + b
            gwait(i, b)
            sissue(i, b)

            @pl.when(i + nbuf < NB)
            def _(i=i, b=b):
                swait(i, b)
                gissue(i + nbuf, b)

    for b in range(nbuf):
        swait(NB - nbuf + b, b)
    plsc.subcore_barrier()

    @pl.when(c == 0)
    def _():
        pltpu.sync_copy(acc.at[pl.ds(r0, ROWS_PER_SUB)],
                        out0.at[pl.ds(r0, ROWS_PER_SUB)])

    @pl.when(c == 1)
    def _():
        pltpu.sync_copy(acc.at[pl.ds(r0, ROWS_PER_SUB)],
                        out1.at[pl.ds(r0, ROWS_PER_SUB)])


@functools.lru_cache(maxsize=None)
def _sc_agg_kernel(d2):
    nbuf = {112: 2, 80: 5, 64: 5, 32: 10, 16: 10}[d2]
    return pl.kernel(
        functools.partial(_sc_agg_body, nbuf),
        out_type=(jax.ShapeDtypeStruct((NP, d2), jnp.float32),
                  jax.ShapeDtypeStruct((NP, d2), jnp.float32)),
        mesh=_mesh(),
        compiler_params=_SC_PARAMS,
        scratch_types=(
            [pltpu.VMEM_SHARED((NP, d2), jnp.float32),
             pltpu.VMEM((NB, EB), jnp.int32),
             pltpu.VMEM((NB, EB), jnp.int32)]
            + [pltpu.VMEM((EB, d2), jnp.float32) for _ in range(nbuf)]
            + [pltpu.SemaphoreType.DMA for _ in range(2 * nbuf)]
        ),
    )


_BLK = 256
_GRID = NP // _BLK


def _row_spec(d):
    return pl.BlockSpec((_BLK, d), lambda i: (i, 0))


def _vec_spec(d):
    return pl.BlockSpec((1, d), lambda i: (0, 0))


def _w_spec(din, dout):
    return pl.BlockSpec((din, dout), lambda i: (0, 0))


def _tc_first_body(x, w, deg0, deg1, z0, z1, dis):
    d = deg0[...][:, :1] + deg1[...][:, :1]
    di = lax.rsqrt(jnp.maximum(d, 1.0))
    z = di * jnp.dot(x[...], w[...], preferred_element_type=jnp.float32)
    h = z.shape[1] // 2
    z0[...] = z[:, :h]
    z1[...] = z[:, h:]
    dis[...] = di


@functools.lru_cache(maxsize=None)
def _tc_first_kernel(din, dout):
    h = dout // 2
    return pl.pallas_call(
        _tc_first_body,
        grid=(_GRID,),
        in_specs=[_row_spec(din), _w_spec(din, dout),
                  _row_spec(DW), _row_spec(DW)],
        out_specs=(_row_spec(h), _row_spec(h), _row_spec(1)),
        out_shape=(jax.ShapeDtypeStruct((NP, h), jnp.float32),
                   jax.ShapeDtypeStruct((NP, h), jnp.float32),
                   jax.ShapeDtypeStruct((NP, 1), jnp.float32)),
    )


def _tc_mid_body(eps, s0, s1, z0, z1, dis, b, g, be, rm, rv, w, o0, o1):
    t = jnp.concatenate([s0[...] + z0[...], s1[...] + z1[...]], axis=1)
    pre = dis[...] * t + b[...]
    hrelu = jnp.maximum(pre, 0.0)
    a = g[...] * lax.rsqrt(rv[...] + eps)
    hbn = (hrelu - rm[...]) * a + be[...]
    z = dis[...] * jnp.dot(hbn, w[...], preferred_element_type=jnp.float32)
    h = z.shape[1] // 2
    o0[...] = z[:, :h]
    o1[...] = z[:, h:]


@functools.lru_cache(maxsize=None)
def _tc_mid_kernel(din, dout):
    hin, hout = din // 2, dout // 2
    return pl.pallas_call(
        functools.partial(_tc_mid_body, 1e-5),
        grid=(_GRID,),
        in_specs=[_row_spec(hin), _row_spec(hin), _row_spec(hin),
                  _row_spec(hin), _row_spec(1),
                  _vec_spec(din), _vec_spec(din), _vec_spec(din),
                  _vec_spec(din), _vec_spec(din), _w_spec(din, dout)],
        out_specs=(_row_spec(hout), _row_spec(hout)),
        out_shape=(jax.ShapeDtypeStruct((NP, hout), jnp.float32),
                   jax.ShapeDtypeStruct((NP, hout), jnp.float32)),
    )


def _tc_last_body(ncls, s0, s1, z0, z1, dis, b, o):
    t = jnp.concatenate([s0[...] + z0[...], s1[...] + z1[...]], axis=1)
    t = dis[...] * t + b[...]
    col = lax.broadcasted_iota(jnp.int32, t.shape, 1)
    mask = col < ncls
    neg = jnp.float32(-3.4e38)
    m = jnp.max(jnp.where(mask, t, neg), axis=1, keepdims=True)
    ex = jnp.where(mask, jnp.exp(t - m), 0.0)
    lse = jnp.log(jnp.sum(ex, axis=1, keepdims=True))
    o[...] = t - m - lse


@functools.lru_cache(maxsize=None)
def _tc_last_kernel(din, ncls):
    hin = din // 2
    return pl.pallas_call(
        functools.partial(_tc_last_body, ncls),
        grid=(_GRID,),
        in_specs=[_row_spec(hin), _row_spec(hin), _row_spec(hin),
                  _row_spec(hin), _row_spec(1), _vec_spec(din)],
        out_specs=_row_spec(din),
        out_shape=jax.ShapeDtypeStruct((NP, din), jnp.float32),
    )


def _pad2(a, r, c):
    return jnp.pad(a, ((0, r - a.shape[0]), (0, c - a.shape[1])))


def _padv(a, c, value=0.0):
    return jnp.pad(a, (0, c - a.shape[0]), constant_values=value)[None, :]


def kernel(x, edge_index, W1, b1, W2, b2, W3, b3, W4, b4, W5, b5,
           g1, be1, rm1, rv1, g2, be2, rm2, rv2, g3, be3, rm3, rv3,
           g4, be4, rm4, rv4):
    src = edge_index[0]
    dst = edge_index[1]
    src3 = src.reshape(NSUB, NB, EB)
    dst3 = dst.reshape(NSUB, NB, EB)

    Ws = [W1, W2, W3, W4, W5]
    bs = [b1, b2, b3, b4, b5]
    gs = [g1, g2, g3, g4]
    bes = [be1, be2, be3, be4]
    rms = [rm1, rm2, rm3, rm4]
    rvs = [rv1, rv2, rv3, rv4]

    xp = _pad2(x, NP, DIMS_P[0])
    wp = [_pad2(Ws[i], DIMS_P[i], DIMS_P[i + 1]) for i in range(5)]
    bp = [_padv(bs[i], DIMS_P[i + 1]) for i in range(5)]
    gp = [_padv(gs[i], DIMS_P[i + 1]) for i in range(4)]
    bep = [_padv(bes[i], DIMS_P[i + 1]) for i in range(4)]
    rmp = [_padv(rms[i], DIMS_P[i + 1]) for i in range(4)]
    rvp = [_padv(rvs[i], DIMS_P[i + 1], 1.0) for i in range(4)]

    zeros1 = jnp.zeros((ROWS_PER_SUB, DW), jnp.float32)
    ones1 = jnp.ones((EB, DW), jnp.float32)

    deg0, deg1 = _sc_degree_kernel()(dst, ones1, zeros1)
    z0, z1, dis = _tc_first_kernel(DIMS_P[0], DIMS_P[1])(xp, wp[0], deg0, deg1)

    for l in range(1, 5):
        d2 = DIMS_P[l] // 2
        zer = jnp.zeros((ROWS_PER_SUB, d2), jnp.float32)
        s0, s1 = _sc_agg_kernel(d2)(src3, dst3, z0, z1, zer)
        z0, z1 = _tc_mid_kernel(DIMS_P[l], DIMS_P[l + 1])(
            s0, s1, z0, z1, dis, bp[l - 1], gp[l - 1], bep[l - 1],
            rmp[l - 1], rvp[l - 1], wp[l])

    d2 = DIMS_P[5] // 2
    zer = jnp.zeros((ROWS_PER_SUB, d2), jnp.float32)
    s0, s1 = _sc_agg_kernel(d2)(src3, dst3, z0, z1, zer)
    out = _tc_last_kernel(DIMS_P[5], 17)(s0, s1, z0, z1, dis, bp[4])
    return out[:N, :17]


# speedup vs baseline: 21.7745x; 1.0672x over previous
import functools

import jax
import jax.numpy as jnp
from jax import lax
from jax.experimental import pallas as pl
from jax.experimental.pallas import tpu as pltpu
from jax.experimental.pallas import tpu_sc as plsc

N = 10000
NP = 10240
E = 320000
NSC = 2
NSUB = 16
EB = 80

DIMS_P = [128, 224, 160, 128, 64, 32]
ROWS_PER_SUB = NP // NSUB
DW = 16
NB = (E // NSUB) // EB
NCH = 5
NBC = NB // NCH

_mesh = functools.partial(
    plsc.VectorSubcoreMesh,
    core_axis_name="c", subcore_axis_name="s",
    num_cores=NSC, num_subcores=NSUB,
)

_SC_PARAMS = pltpu.CompilerParams(use_tc_tiling_on_sc=False)


_DEG_NB = (E // (NSC * NSUB)) // EB
_DEG_RING = 8


def _sc_degree_body(dst3, ones, zeros, deg0, deg1, acc, didx, obuf, *ssems):
    s = lax.axis_index("s")
    c = lax.axis_index("c")
    r0 = s * ROWS_PER_SUB
    pltpu.sync_copy(zeros, acc.at[pl.ds(r0, ROWS_PER_SUB)])
    pltpu.sync_copy(ones, obuf)
    w = s * NSC + c
    pltpu.sync_copy(dst3.at[w], didx)
    plsc.subcore_barrier()

    def sissue(i, b):
        pltpu.async_copy(obuf, acc.at[didx.at[i]], ssems[b], add=True)

    def swait(i, b):
        pltpu.make_async_copy(obuf, acc.at[didx.at[i]], ssems[b]).wait()

    for b in range(_DEG_RING):
        sissue(b, b)

    @pl.loop(0, _DEG_NB // _DEG_RING - 1)
    def _segments(j):
        i0 = j * _DEG_RING
        for b in range(_DEG_RING):
            swait(i0 + b, b)
            sissue(i0 + _DEG_RING + b, b)

    i0 = _DEG_NB - _DEG_RING - (_DEG_NB % _DEG_RING)
    for k in range(_DEG_NB % _DEG_RING):
        b = k % _DEG_RING
        swait(i0 + k, b)
        sissue(i0 + _DEG_RING + k, b)
    for k in range(_DEG_RING):
        i = _DEG_NB - _DEG_RING + k
        swait(i, i % _DEG_RING)

    plsc.subcore_barrier()

    @pl.when(c == 0)
    def _():
        pltpu.sync_copy(acc.at[pl.ds(r0, ROWS_PER_SUB)],
                        deg0.at[pl.ds(r0, ROWS_PER_SUB)])

    @pl.when(c == 1)
    def _():
        pltpu.sync_copy(acc.at[pl.ds(r0, ROWS_PER_SUB)],
                        deg1.at[pl.ds(r0, ROWS_PER_SUB)])


@functools.lru_cache(maxsize=None)
def _sc_degree_kernel():
    return pl.kernel(
        _sc_degree_body,
        out_type=(jax.ShapeDtypeStruct((NP, DW), jnp.float32),
                  jax.ShapeDtypeStruct((NP, DW), jnp.float32)),
        mesh=_mesh(),
        compiler_params=_SC_PARAMS,
        scratch_types=[
            pltpu.VMEM_SHARED((NP, DW), jnp.float32),
            pltpu.VMEM((_DEG_NB, EB), jnp.int32),
            pltpu.VMEM((EB, DW), jnp.float32),
        ] + [pltpu.SemaphoreType.DMA for _ in range(_DEG_RING)],
    )


def _sc_agg_body(nbuf, src4, dst4, z0, z1, zeros, out0, out1,
                 acc, sidx, didx, *bufs_and_sems):
    s = lax.axis_index("s")
    c = lax.axis_index("c")
    r0 = s * ROWS_PER_SUB
    ebufs = bufs_and_sems[:nbuf]
    gsems = bufs_and_sems[nbuf:2 * nbuf]
    ssems = bufs_and_sems[2 * nbuf:]

    pltpu.sync_copy(zeros, acc.at[pl.ds(r0, ROWS_PER_SUB)])

    def gissue(i, b):
        @pl.when(c == 0)
        def _():
            pltpu.async_copy(z0.at[sidx.at[i]], ebufs[b], gsems[b])

        @pl.when(c == 1)
        def _():
            pltpu.async_copy(z1.at[sidx.at[i]], ebufs[b], gsems[b])

    def gwait(i, b):
        @pl.when(c == 0)
        def _():
            pltpu.make_async_copy(z0.at[sidx.at[i]], ebufs[b], gsems[b]).wait()

        @pl.when(c == 1)
        def _():
            pltpu.make_async_copy(z1.at[sidx.at[i]], ebufs[b], gsems[b]).wait()

    def sissue(i, b):
        pltpu.async_copy(ebufs[b], acc.at[didx.at[i]], ssems[b], add=True)

    def swait(i, b):
        pltpu.make_async_copy(ebufs[b], acc.at[didx.at[i]], ssems[b]).wait()

    plsc.subcore_barrier()

    @pl.loop(0, NCH)
    def _chunk(ch):
        pltpu.sync_copy(src4.at[s, ch], sidx)
        pltpu.sync_copy(dst4.at[s, ch], didx)
        for b in range(nbuf):
            gissue(b, b)

        @pl.loop(0, NBC // nbuf)
        def _segments(j):
            i0 = j * nbuf
            for b in range(nbuf):
                i = i0 + b
                gwait(i, b)
                sissue(i, b)

                @pl.when(i + nbuf < NBC)
                def _(i=i, b=b):
                    swait(i, b)
                    gissue(i + nbuf, b)

        for b in range(nbuf):
            swait(NBC - nbuf + b, b)

    plsc.subcore_barrier()

    @pl.when(c == 0)
    def _():
        pltpu.sync_copy(acc.at[pl.ds(r0, ROWS_PER_SUB)],
                        out0.at[pl.ds(r0, ROWS_PER_SUB)])

    @pl.when(c == 1)
    def _():
        pltpu.sync_copy(acc.at[pl.ds(r0, ROWS_PER_SUB)],
                        out1.at[pl.ds(r0, ROWS_PER_SUB)])


@functools.lru_cache(maxsize=None)
def _sc_agg_kernel(d2):
    nbuf = {112: 5, 80: 10, 64: 10, 32: 10, 16: 10}[d2]
    return pl.kernel(
        functools.partial(_sc_agg_body, nbuf),
        out_type=(jax.ShapeDtypeStruct((NP, d2), jnp.float32),
                  jax.ShapeDtypeStruct((NP, d2), jnp.float32)),
        mesh=_mesh(),
        compiler_params=_SC_PARAMS,
        scratch_types=(
            [pltpu.VMEM_SHARED((NP, d2), jnp.float32),
             pltpu.VMEM((NBC, EB), jnp.int32),
             pltpu.VMEM((NBC, EB), jnp.int32)]
            + [pltpu.VMEM((EB, d2), jnp.float32) for _ in range(nbuf)]
            + [pltpu.SemaphoreType.DMA for _ in range(2 * nbuf)]
        ),
    )


_BLK = 256
_GRID = NP // _BLK


def _row_spec(d):
    return pl.BlockSpec((_BLK, d), lambda i: (i, 0))


def _vec_spec(d):
    return pl.BlockSpec((1, d), lambda i: (0, 0))


def _w_spec(din, dout):
    return pl.BlockSpec((din, dout), lambda i: (0, 0))


def _tc_first_body(x, w, deg0, deg1, z0, z1, dis):
    d = deg0[...][:, :1] + deg1[...][:, :1]
    di = lax.rsqrt(jnp.maximum(d, 1.0))
    z = di * jnp.dot(x[...], w[...], preferred_element_type=jnp.float32)
    h = z.shape[1] // 2
    z0[...] = z[:, :h]
    z1[...] = z[:, h:]
    dis[...] = di


@functools.lru_cache(maxsize=None)
def _tc_first_kernel(din, dout):
    h = dout // 2
    return pl.pallas_call(
        _tc_first_body,
        grid=(_GRID,),
        in_specs=[_row_spec(din), _w_spec(din, dout),
                  _row_spec(DW), _row_spec(DW)],
        out_specs=(_row_spec(h), _row_spec(h), _row_spec(1)),
        out_shape=(jax.ShapeDtypeStruct((NP, h), jnp.float32),
                   jax.ShapeDtypeStruct((NP, h), jnp.float32),
                   jax.ShapeDtypeStruct((NP, 1), jnp.float32)),
    )


def _tc_mid_body(eps, s0, s1, z0, z1, dis, b, g, be, rm, rv, w, o0, o1):
    t = jnp.concatenate([s0[...] + z0[...], s1[...] + z1[...]], axis=1)
    pre = dis[...] * t + b[...]
    hrelu = jnp.maximum(pre, 0.0)
    a = g[...] * lax.rsqrt(rv[...] + eps)
    hbn = (hrelu - rm[...]) * a + be[...]
    z = dis[...] * jnp.dot(hbn, w[...], preferred_element_type=jnp.float32)
    h = z.shape[1] // 2
    o0[...] = z[:, :h]
    o1[...] = z[:, h:]


@functools.lru_cache(maxsize=None)
def _tc_mid_kernel(din, dout):
    hin, hout = din // 2, dout // 2
    return pl.pallas_call(
        functools.partial(_tc_mid_body, 1e-5),
        grid=(_GRID,),
        in_specs=[_row_spec(hin), _row_spec(hin), _row_spec(hin),
                  _row_spec(hin), _row_spec(1),
                  _vec_spec(din), _vec_spec(din), _vec_spec(din),
                  _vec_spec(din), _vec_spec(din), _w_spec(din, dout)],
        out_specs=(_row_spec(hout), _row_spec(hout)),
        out_shape=(jax.ShapeDtypeStruct((NP, hout), jnp.float32),
                   jax.ShapeDtypeStruct((NP, hout), jnp.float32)),
    )


def _tc_last_body(ncls, s0, s1, z0, z1, dis, b, o):
    t = jnp.concatenate([s0[...] + z0[...], s1[...] + z1[...]], axis=1)
    t = dis[...] * t + b[...]
    col = lax.broadcasted_iota(jnp.int32, t.shape, 1)
    mask = col < ncls
    neg = jnp.float32(-3.4e38)
    m = jnp.max(jnp.where(mask, t, neg), axis=1, keepdims=True)
    ex = jnp.where(mask, jnp.exp(t - m), 0.0)
    lse = jnp.log(jnp.sum(ex, axis=1, keepdims=True))
    o[...] = t - m - lse


@functools.lru_cache(maxsize=None)
def _tc_last_kernel(din, ncls):
    hin = din // 2
    return pl.pallas_call(
        functools.partial(_tc_last_body, ncls),
        grid=(_GRID,),
        in_specs=[_row_spec(hin), _row_spec(hin), _row_spec(hin),
                  _row_spec(hin), _row_spec(1), _vec_spec(din)],
        out_specs=_row_spec(din),
        out_shape=jax.ShapeDtypeStruct((NP, din), jnp.float32),
    )


def _pad2(a, r, c):
    return jnp.pad(a, ((0, r - a.shape[0]), (0, c - a.shape[1])))


def _padv(a, c, value=0.0):
    return jnp.pad(a, (0, c - a.shape[0]), constant_values=value)[None, :]


def kernel(x, edge_index, W1, b1, W2, b2, W3, b3, W4, b4, W5, b5,
           g1, be1, rm1, rv1, g2, be2, rm2, rv2, g3, be3, rm3, rv3,
           g4, be4, rm4, rv4):
    src = edge_index[0]
    dst = edge_index[1]
    src4 = src.reshape(NSUB, NCH, NBC, EB)
    dst4 = dst.reshape(NSUB, NCH, NBC, EB)

    Ws = [W1, W2, W3, W4, W5]
    bs = [b1, b2, b3, b4, b5]
    gs = [g1, g2, g3, g4]
    bes = [be1, be2, be3, be4]
    rms = [rm1, rm2, rm3, rm4]
    rvs = [rv1, rv2, rv3, rv4]

    xp = _pad2(x, NP, DIMS_P[0])
    wp = [_pad2(Ws[i], DIMS_P[i], DIMS_P[i + 1]) for i in range(5)]
    bp = [_padv(bs[i], DIMS_P[i + 1]) for i in range(5)]
    gp = [_padv(gs[i], DIMS_P[i + 1]) for i in range(4)]
    bep = [_padv(bes[i], DIMS_P[i + 1]) for i in range(4)]
    rmp = [_padv(rms[i], DIMS_P[i + 1]) for i in range(4)]
    rvp = [_padv(rvs[i], DIMS_P[i + 1], 1.0) for i in range(4)]

    zeros1 = jnp.zeros((ROWS_PER_SUB, DW), jnp.float32)
    ones1 = jnp.ones((EB, DW), jnp.float32)

    dstw = dst.reshape(NSUB * NSC, _DEG_NB, EB)
    deg0, deg1 = _sc_degree_kernel()(dstw, ones1, zeros1)
    z0, z1, dis = _tc_first_kernel(DIMS_P[0], DIMS_P[1])(xp, wp[0], deg0, deg1)

    for l in range(1, 5):
        d2 = DIMS_P[l] // 2
        zer = jnp.zeros((ROWS_PER_SUB, d2), jnp.float32)
        s0, s1 = _sc_agg_kernel(d2)(src4, dst4, z0, z1, zer)
        z0, z1 = _tc_mid_kernel(DIMS_P[l], DIMS_P[l + 1])(
            s0, s1, z0, z1, dis, bp[l - 1], gp[l - 1], bep[l - 1],
            rmp[l - 1], rvp[l - 1], wp[l])

    d2 = DIMS_P[5] // 2
    zer = jnp.zeros((ROWS_PER_SUB, d2), jnp.float32)
    s0, s1 = _sc_agg_kernel(d2)(src4, dst4, z0, z1, zer)
    out = _tc_last_kernel(DIMS_P[5], 17)(s0, s1, z0, z1, dis, bp[4])
    return out[:N, :17]


# speedup vs baseline: 21.7813x; 1.0003x over previous
import functools

import jax
import jax.numpy as jnp
from jax import lax
from jax.experimental import pallas as pl
from jax.experimental.pallas import tpu as pltpu
from jax.experimental.pallas import tpu_sc as plsc

N = 10000
NP = 10240
E = 320000
NSC = 2
NSUB = 16
EB = 80

DIMS_P = [128, 224, 160, 128, 64, 32]
ROWS_PER_SUB = NP // NSUB
DW = 16
NB = (E // NSUB) // EB
NCH = 5
NBC = NB // NCH

_mesh = functools.partial(
    plsc.VectorSubcoreMesh,
    core_axis_name="c", subcore_axis_name="s",
    num_cores=NSC, num_subcores=NSUB,
)

_SC_PARAMS = pltpu.CompilerParams(use_tc_tiling_on_sc=False)


_DEG_NB = (E // (NSC * NSUB)) // EB
_DEG_RING = 8


def _sc_degree_body(dst3, ones, zeros, deg0, deg1, acc, didx, obuf, *ssems):
    s = lax.axis_index("s")
    c = lax.axis_index("c")
    r0 = s * ROWS_PER_SUB
    pltpu.sync_copy(zeros, acc.at[pl.ds(r0, ROWS_PER_SUB)])
    pltpu.sync_copy(ones, obuf)
    w = s * NSC + c
    pltpu.sync_copy(dst3.at[w], didx)
    plsc.subcore_barrier()

    def sissue(i, b):
        pltpu.async_copy(obuf, acc.at[didx.at[i]], ssems[b], add=True)

    def swait(i, b):
        pltpu.make_async_copy(obuf, acc.at[didx.at[i]], ssems[b]).wait()

    for b in range(_DEG_RING):
        sissue(b, b)

    @pl.loop(0, _DEG_NB // _DEG_RING - 1)
    def _segments(j):
        i0 = j * _DEG_RING
        for b in range(_DEG_RING):
            swait(i0 + b, b)
            sissue(i0 + _DEG_RING + b, b)

    i0 = _DEG_NB - _DEG_RING - (_DEG_NB % _DEG_RING)
    for k in range(_DEG_NB % _DEG_RING):
        b = k % _DEG_RING
        swait(i0 + k, b)
        sissue(i0 + _DEG_RING + k, b)
    for k in range(_DEG_RING):
        i = _DEG_NB - _DEG_RING + k
        swait(i, i % _DEG_RING)

    plsc.subcore_barrier()

    @pl.when(c == 0)
    def _():
        pltpu.sync_copy(acc.at[pl.ds(r0, ROWS_PER_SUB)],
                        deg0.at[pl.ds(r0, ROWS_PER_SUB)])

    @pl.when(c == 1)
    def _():
        pltpu.sync_copy(acc.at[pl.ds(r0, ROWS_PER_SUB)],
                        deg1.at[pl.ds(r0, ROWS_PER_SUB)])


@functools.lru_cache(maxsize=None)
def _sc_degree_kernel():
    return pl.kernel(
        _sc_degree_body,
        out_type=(jax.ShapeDtypeStruct((NP, DW), jnp.float32),
                  jax.ShapeDtypeStruct((NP, DW), jnp.float32)),
        mesh=_mesh(),
        compiler_params=_SC_PARAMS,
        scratch_types=[
            pltpu.VMEM_SHARED((NP, DW), jnp.float32),
            pltpu.VMEM((_DEG_NB, EB), jnp.int32),
            pltpu.VMEM((EB, DW), jnp.float32),
        ] + [pltpu.SemaphoreType.DMA for _ in range(_DEG_RING)],
    )


def _sc_agg_body(nbuf, src4, dst4, z0, z1, zeros, out0, out1,
                 acc, sidx, didx, *bufs_and_sems):
    s = lax.axis_index("s")
    c = lax.axis_index("c")
    r0 = s * ROWS_PER_SUB
    ebufs = bufs_and_sems[:nbuf]
    gsems = bufs_and_sems[nbuf:2 * nbuf]
    ssems = bufs_and_sems[2 * nbuf:]

    pltpu.sync_copy(zeros, acc.at[pl.ds(r0, ROWS_PER_SUB)])

    def gissue(i, b):
        @pl.when(c == 0)
        def _():
            pltpu.async_copy(z0.at[sidx.at[i]], ebufs[b], gsems[b])

        @pl.when(c == 1)
        def _():
            pltpu.async_copy(z1.at[sidx.at[i]], ebufs[b], gsems[b])

    def gwait(i, b):
        @pl.when(c == 0)
        def _():
            pltpu.make_async_copy(z0.at[sidx.at[i]], ebufs[b], gsems[b]).wait()

        @pl.when(c == 1)
        def _():
            pltpu.make_async_copy(z1.at[sidx.at[i]], ebufs[b], gsems[b]).wait()

    def sissue(i, b):
        pltpu.async_copy(ebufs[b], acc.at[didx.at[i]], ssems[b], add=True)

    def swait(i, b):
        pltpu.make_async_copy(ebufs[b], acc.at[didx.at[i]], ssems[b]).wait()

    plsc.subcore_barrier()

    lag = 2

    @pl.loop(0, NCH)
    def _chunk(ch):
        pltpu.sync_copy(src4.at[s, ch], sidx)
        pltpu.sync_copy(dst4.at[s, ch], didx)
        for b in range(nbuf - lag):
            gissue(b, b)
        for i in range(nbuf):
            gwait(i, i)
            sissue(i, i)
            if i < lag:
                gissue(i + nbuf - lag, i + nbuf - lag)
            else:
                swait(i - lag, (i - lag) % nbuf)
                gissue(i + nbuf - lag, (i + nbuf - lag) % nbuf)

        @pl.loop(1, NBC // nbuf)
        def _segments(j):
            i0 = j * nbuf
            for b in range(nbuf):
                i = i0 + b
                gwait(i, b)
                sissue(i, b)
                swait(i - lag, (b - lag) % nbuf)

                @pl.when(i + nbuf - lag < NBC)
                def _(i=i, b=b):
                    gissue(i + nbuf - lag, (b + nbuf - lag) % nbuf)

        for k in range(lag):
            swait(NBC - lag + k, (NBC - lag + k) % nbuf)

    plsc.subcore_barrier()

    @pl.when(c == 0)
    def _():
        pltpu.sync_copy(acc.at[pl.ds(r0, ROWS_PER_SUB)],
                        out0.at[pl.ds(r0, ROWS_PER_SUB)])

    @pl.when(c == 1)
    def _():
        pltpu.sync_copy(acc.at[pl.ds(r0, ROWS_PER_SUB)],
                        out1.at[pl.ds(r0, ROWS_PER_SUB)])


@functools.lru_cache(maxsize=None)
def _sc_agg_kernel(d2):
    nbuf = {112: 5, 80: 10, 64: 10, 32: 10, 16: 10}[d2]
    return pl.kernel(
        functools.partial(_sc_agg_body, nbuf),
        out_type=(jax.ShapeDtypeStruct((NP, d2), jnp.float32),
                  jax.ShapeDtypeStruct((NP, d2), jnp.float32)),
        mesh=_mesh(),
        compiler_params=_SC_PARAMS,
        scratch_types=(
            [pltpu.VMEM_SHARED((NP, d2), jnp.float32),
             pltpu.VMEM((NBC, EB), jnp.int32),
             pltpu.VMEM((NBC, EB), jnp.int32)]
            + [pltpu.VMEM((EB, d2), jnp.float32) for _ in range(nbuf)]
            + [pltpu.SemaphoreType.DMA for _ in range(2 * nbuf)]
        ),
    )


_BLK = 256
_GRID = NP // _BLK


def _row_spec(d):
    return pl.BlockSpec((_BLK, d), lambda i: (i, 0))


def _vec_spec(d):
    return pl.BlockSpec((1, d), lambda i: (0, 0))


def _w_spec(din, dout):
    return pl.BlockSpec((din, dout), lambda i: (0, 0))


def _tc_first_body(x, w, deg0, deg1, z0, z1, dis):
    d = deg0[...][:, :1] + deg1[...][:, :1]
    di = lax.rsqrt(jnp.maximum(d, 1.0))
    z = di * jnp.dot(x[...], w[...], preferred_element_type=jnp.float32)
    h = z.shape[1] // 2
    z0[...] = z[:, :h]
    z1[...] = z[:, h:]
    dis[...] = di


@functools.lru_cache(maxsize=None)
def _tc_first_kernel(din, dout):
    h = dout // 2
    return pl.pallas_call(
        _tc_first_body,
        grid=(_GRID,),
        in_specs=[_row_spec(din), _w_spec(din, dout),
                  _row_spec(DW), _row_spec(DW)],
        out_specs=(_row_spec(h), _row_spec(h), _row_spec(1)),
        out_shape=(jax.ShapeDtypeStruct((NP, h), jnp.float32),
                   jax.ShapeDtypeStruct((NP, h), jnp.float32),
                   jax.ShapeDtypeStruct((NP, 1), jnp.float32)),
    )


def _tc_mid_body(eps, s0, s1, z0, z1, dis, b, g, be, rm, rv, w, o0, o1):
    t = jnp.concatenate([s0[...] + z0[...], s1[...] + z1[...]], axis=1)
    pre = dis[...] * t + b[...]
    hrelu = jnp.maximum(pre, 0.0)
    a = g[...] * lax.rsqrt(rv[...] + eps)
    hbn = (hrelu - rm[...]) * a + be[...]
    z = dis[...] * jnp.dot(hbn, w[...], preferred_element_type=jnp.float32)
    h = z.shape[1] // 2
    o0[...] = z[:, :h]
    o1[...] = z[:, h:]


@functools.lru_cache(maxsize=None)
def _tc_mid_kernel(din, dout):
    hin, hout = din // 2, dout // 2
    return pl.pallas_call(
        functools.partial(_tc_mid_body, 1e-5),
        grid=(_GRID,),
        in_specs=[_row_spec(hin), _row_spec(hin), _row_spec(hin),
                  _row_spec(hin), _row_spec(1),
                  _vec_spec(din), _vec_spec(din), _vec_spec(din),
                  _vec_spec(din), _vec_spec(din), _w_spec(din, dout)],
        out_specs=(_row_spec(hout), _row_spec(hout)),
        out_shape=(jax.ShapeDtypeStruct((NP, hout), jnp.float32),
                   jax.ShapeDtypeStruct((NP, hout), jnp.float32)),
    )


def _tc_last_body(ncls, s0, s1, z0, z1, dis, b, o):
    t = jnp.concatenate([s0[...] + z0[...], s1[...] + z1[...]], axis=1)
    t = dis[...] * t + b[...]
    col = lax.broadcasted_iota(jnp.int32, t.shape, 1)
    mask = col < ncls
    neg = jnp.float32(-3.4e38)
    m = jnp.max(jnp.where(mask, t, neg), axis=1, keepdims=True)
    ex = jnp.where(mask, jnp.exp(t - m), 0.0)
    lse = jnp.log(jnp.sum(ex, axis=1, keepdims=True))
    o[...] = t - m - lse


@functools.lru_cache(maxsize=None)
def _tc_last_kernel(din, ncls):
    hin = din // 2
    return pl.pallas_call(
        functools.partial(_tc_last_body, ncls),
        grid=(_GRID,),
        in_specs=[_row_spec(hin), _row_spec(hin), _row_spec(hin),
                  _row_spec(hin), _row_spec(1), _vec_spec(din)],
        out_specs=_row_spec(din),
        out_shape=jax.ShapeDtypeStruct((NP, din), jnp.float32),
    )


def _pad2(a, r, c):
    return jnp.pad(a, ((0, r - a.shape[0]), (0, c - a.shape[1])))


def _padv(a, c, value=0.0):
    return jnp.pad(a, (0, c - a.shape[0]), constant_values=value)[None, :]


def kernel(x, edge_index, W1, b1, W2, b2, W3, b3, W4, b4, W5, b5,
           g1, be1, rm1, rv1, g2, be2, rm2, rv2, g3, be3, rm3, rv3,
           g4, be4, rm4, rv4):
    src = edge_index[0]
    dst = edge_index[1]
    src4 = src.reshape(NSUB, NCH, NBC, EB)
    dst4 = dst.reshape(NSUB, NCH, NBC, EB)

    Ws = [W1, W2, W3, W4, W5]
    bs = [b1, b2, b3, b4, b5]
    gs = [g1, g2, g3, g4]
    bes = [be1, be2, be3, be4]
    rms = [rm1, rm2, rm3, rm4]
    rvs = [rv1, rv2, rv3, rv4]

    xp = _pad2(x, NP, DIMS_P[0])
    wp = [_pad2(Ws[i], DIMS_P[i], DIMS_P[i + 1]) for i in range(5)]
    bp = [_padv(bs[i], DIMS_P[i + 1]) for i in range(5)]
    gp = [_padv(gs[i], DIMS_P[i + 1]) for i in range(4)]
    bep = [_padv(bes[i], DIMS_P[i + 1]) for i in range(4)]
    rmp = [_padv(rms[i], DIMS_P[i + 1]) for i in range(4)]
    rvp = [_padv(rvs[i], DIMS_P[i + 1], 1.0) for i in range(4)]

    zeros1 = jnp.zeros((ROWS_PER_SUB, DW), jnp.float32)
    ones1 = jnp.ones((EB, DW), jnp.float32)

    dstw = dst.reshape(NSUB * NSC, _DEG_NB, EB)
    deg0, deg1 = _sc_degree_kernel()(dstw, ones1, zeros1)
    z0, z1, dis = _tc_first_kernel(DIMS_P[0], DIMS_P[1])(xp, wp[0], deg0, deg1)

    for l in range(1, 5):
        d2 = DIMS_P[l] // 2
        zer = jnp.zeros((ROWS_PER_SUB, d2), jnp.float32)
        s0, s1 = _sc_agg_kernel(d2)(src4, dst4, z0, z1, zer)
        z0, z1 = _tc_mid_kernel(DIMS_P[l], DIMS_P[l + 1])(
            s0, s1, z0, z1, dis, bp[l - 1], gp[l - 1], bep[l - 1],
            rmp[l - 1], rvp[l - 1], wp[l])

    d2 = DIMS_P[5] // 2
    zer = jnp.zeros((ROWS_PER_SUB, d2), jnp.float32)
    s0, s1 = _sc_agg_kernel(d2)(src4, dst4, z0, z1, zer)
    out = _tc_last_kernel(DIMS_P[5], 17)(s0, s1, z0, z1, dis, bp[4])
    return out[:N, :17]


# speedup vs baseline: 23.9561x; 1.0998x over previous
import functools

import jax
import jax.numpy as jnp
from jax import lax
from jax.experimental import pallas as pl
from jax.experimental.pallas import tpu as pltpu
from jax.experimental.pallas import tpu_sc as plsc

N = 10000
NP = 10240
E = 320000
NSC = 2
NSUB = 16
EB = 80

DIMS_P = [128, 224, 160, 128, 64, 32]
ROWS_PER_SUB = NP // NSUB
DW = 16
NB = (E // NSUB) // EB
NCH = 5
NBC = NB // NCH

_mesh = functools.partial(
    plsc.VectorSubcoreMesh,
    core_axis_name="c", subcore_axis_name="s",
    num_cores=NSC, num_subcores=NSUB,
)

_SC_PARAMS = pltpu.CompilerParams(use_tc_tiling_on_sc=False)


_DEG_NB = (E // (NSC * NSUB)) // EB
_DEG_RING = 8


def _sc_degree_body(dst3, ones, zeros, deg0, deg1, acc, didx, obuf, *ssems):
    s = lax.axis_index("s")
    c = lax.axis_index("c")
    r0 = s * ROWS_PER_SUB
    pltpu.sync_copy(zeros, acc.at[pl.ds(r0, ROWS_PER_SUB)])
    pltpu.sync_copy(ones, obuf)
    w = s * NSC + c
    pltpu.sync_copy(dst3.at[w], didx)
    plsc.subcore_barrier()

    def sissue(i, b):
        pltpu.async_copy(obuf, acc.at[didx.at[i]], ssems[b], add=True)

    def swait(i, b):
        pltpu.make_async_copy(obuf, acc.at[didx.at[i]], ssems[b]).wait()

    for b in range(_DEG_RING):
        sissue(b, b)

    @pl.loop(0, _DEG_NB // _DEG_RING - 1)
    def _segments(j):
        i0 = j * _DEG_RING
        for b in range(_DEG_RING):
            swait(i0 + b, b)
            sissue(i0 + _DEG_RING + b, b)

    i0 = _DEG_NB - _DEG_RING - (_DEG_NB % _DEG_RING)
    for k in range(_DEG_NB % _DEG_RING):
        b = k % _DEG_RING
        swait(i0 + k, b)
        sissue(i0 + _DEG_RING + k, b)
    for k in range(_DEG_RING):
        i = _DEG_NB - _DEG_RING + k
        swait(i, i % _DEG_RING)

    plsc.subcore_barrier()

    @pl.when(c == 0)
    def _():
        pltpu.sync_copy(acc.at[pl.ds(r0, ROWS_PER_SUB)],
                        deg0.at[pl.ds(r0, ROWS_PER_SUB)])

    @pl.when(c == 1)
    def _():
        pltpu.sync_copy(acc.at[pl.ds(r0, ROWS_PER_SUB)],
                        deg1.at[pl.ds(r0, ROWS_PER_SUB)])


@functools.lru_cache(maxsize=None)
def _sc_degree_kernel():
    return pl.kernel(
        _sc_degree_body,
        out_type=(jax.ShapeDtypeStruct((NP, DW), jnp.float32),
                  jax.ShapeDtypeStruct((NP, DW), jnp.float32)),
        mesh=_mesh(),
        compiler_params=_SC_PARAMS,
        scratch_types=[
            pltpu.VMEM_SHARED((NP, DW), jnp.float32),
            pltpu.VMEM((_DEG_NB, EB), jnp.int32),
            pltpu.VMEM((EB, DW), jnp.float32),
        ] + [pltpu.SemaphoreType.DMA for _ in range(_DEG_RING)],
    )


def _sc_agg_body(nbuf, src4, dst4, z0, z1, zeros, out0, out1,
                 acc, sidx, didx, *bufs_and_sems):
    s = lax.axis_index("s")
    c = lax.axis_index("c")
    r0 = s * ROWS_PER_SUB
    ebufs = bufs_and_sems[:nbuf]
    gsems = bufs_and_sems[nbuf:2 * nbuf]
    ssems = bufs_and_sems[2 * nbuf:]

    pltpu.sync_copy(zeros, acc.at[pl.ds(r0, ROWS_PER_SUB)])

    def gissue(i, b):
        @pl.when(c == 0)
        def _():
            pltpu.async_copy(z0.at[sidx.at[i]], ebufs[b], gsems[b])

        @pl.when(c == 1)
        def _():
            pltpu.async_copy(z1.at[sidx.at[i]], ebufs[b], gsems[b])

    def gwait(i, b):
        @pl.when(c == 0)
        def _():
            pltpu.make_async_copy(z0.at[sidx.at[i]], ebufs[b], gsems[b]).wait()

        @pl.when(c == 1)
        def _():
            pltpu.make_async_copy(z1.at[sidx.at[i]], ebufs[b], gsems[b]).wait()

    def sissue(i, b):
        pltpu.async_copy(ebufs[b], acc.at[didx.at[i]], ssems[b], add=True)

    def swait(i, b):
        pltpu.make_async_copy(ebufs[b], acc.at[didx.at[i]], ssems[b]).wait()

    plsc.subcore_barrier()

    lag = 2

    @pl.loop(0, NCH)
    def _chunk(ch):
        pltpu.sync_copy(src4.at[s, ch], sidx)
        pltpu.sync_copy(dst4.at[s, ch], didx)
        for b in range(nbuf - lag):
            gissue(b, b)
        for i in range(nbuf):
            gwait(i, i)
            sissue(i, i)
            if i < lag:
                gissue(i + nbuf - lag, i + nbuf - lag)
            else:
                swait(i - lag, (i - lag) % nbuf)
                gissue(i + nbuf - lag, (i + nbuf - lag) % nbuf)

        @pl.loop(1, NBC // nbuf)
        def _segments(j):
            i0 = j * nbuf
            for b in range(nbuf):
                i = i0 + b
                gwait(i, b)
                sissue(i, b)
                swait(i - lag, (b - lag) % nbuf)

                @pl.when(i + nbuf - lag < NBC)
                def _(i=i, b=b):
                    gissue(i + nbuf - lag, (b + nbuf - lag) % nbuf)

        for k in range(lag):
            swait(NBC - lag + k, (NBC - lag + k) % nbuf)

    plsc.subcore_barrier()

    @pl.when(c == 0)
    def _():
        pltpu.sync_copy(acc.at[pl.ds(r0, ROWS_PER_SUB)],
                        out0.at[pl.ds(r0, ROWS_PER_SUB)])

    @pl.when(c == 1)
    def _():
        pltpu.sync_copy(acc.at[pl.ds(r0, ROWS_PER_SUB)],
                        out1.at[pl.ds(r0, ROWS_PER_SUB)])


@functools.lru_cache(maxsize=None)
def _sc_agg_kernel(d2):
    nbuf = {112: 5, 80: 10, 64: 10, 32: 10, 16: 10}[d2]
    return pl.kernel(
        functools.partial(_sc_agg_body, nbuf),
        out_type=(jax.ShapeDtypeStruct((NP, d2), jnp.float32),
                  jax.ShapeDtypeStruct((NP, d2), jnp.float32)),
        mesh=_mesh(),
        compiler_params=_SC_PARAMS,
        scratch_types=(
            [pltpu.VMEM_SHARED((NP, d2), jnp.float32),
             pltpu.VMEM((NBC, EB), jnp.int32),
             pltpu.VMEM((NBC, EB), jnp.int32)]
            + [pltpu.VMEM((EB, d2), jnp.float32) for _ in range(nbuf)]
            + [pltpu.SemaphoreType.DMA for _ in range(2 * nbuf)]
        ),
    )


_BLK = 256
_GRID = NP // _BLK


def _row_spec(d):
    return pl.BlockSpec((_BLK, d), lambda i: (i, 0))


def _vec_spec(d):
    return pl.BlockSpec((1, d), lambda i: (0, 0))


def _w_spec(din, dout):
    return pl.BlockSpec((din, dout), lambda i: (0, 0))


def _tc_first_body(x, deg0, deg1, z0, z1, dis):
    d = deg0[...][:, :1] + deg1[...][:, :1]
    di = lax.rsqrt(jnp.maximum(d, 1.0))
    z = di * x[...]
    h = z.shape[1] // 2
    z0[...] = z[:, :h]
    z1[...] = z[:, h:]
    dis[...] = di


@functools.lru_cache(maxsize=None)
def _tc_first_kernel(din):
    h = din // 2
    return pl.pallas_call(
        _tc_first_body,
        grid=(_GRID,),
        in_specs=[_row_spec(din), _row_spec(DW), _row_spec(DW)],
        out_specs=(_row_spec(h), _row_spec(h), _row_spec(1)),
        out_shape=(jax.ShapeDtypeStruct((NP, h), jnp.float32),
                   jax.ShapeDtypeStruct((NP, h), jnp.float32),
                   jax.ShapeDtypeStruct((NP, 1), jnp.float32)),
    )


def _tc_mid1_body(eps, s0, s1, z0, z1, dis, w1, b, g, be, rm, rv, w2, o0, o1):
    t = jnp.concatenate([s0[...] + z0[...], s1[...] + z1[...]], axis=1)
    u = jnp.dot(dis[...] * t, w1[...],
                preferred_element_type=jnp.float32) + b[...]
    hrelu = jnp.maximum(u, 0.0)
    a = g[...] * lax.rsqrt(rv[...] + eps)
    hbn = (hrelu - rm[...]) * a + be[...]
    z = dis[...] * jnp.dot(hbn, w2[...], preferred_element_type=jnp.float32)
    h = z.shape[1] // 2
    o0[...] = z[:, :h]
    o1[...] = z[:, h:]


@functools.lru_cache(maxsize=None)
def _tc_mid1_kernel(din, dmid, dout):
    hin, hout = din // 2, dout // 2
    return pl.pallas_call(
        functools.partial(_tc_mid1_body, 1e-5),
        grid=(_GRID,),
        in_specs=[_row_spec(hin), _row_spec(hin), _row_spec(hin),
                  _row_spec(hin), _row_spec(1), _w_spec(din, dmid),
                  _vec_spec(dmid), _vec_spec(dmid), _vec_spec(dmid),
                  _vec_spec(dmid), _vec_spec(dmid), _w_spec(dmid, dout)],
        out_specs=(_row_spec(hout), _row_spec(hout)),
        out_shape=(jax.ShapeDtypeStruct((NP, hout), jnp.float32),
                   jax.ShapeDtypeStruct((NP, hout), jnp.float32)),
    )


def _tc_mid_body(eps, s0, s1, z0, z1, dis, b, g, be, rm, rv, w, o0, o1):
    t = jnp.concatenate([s0[...] + z0[...], s1[...] + z1[...]], axis=1)
    pre = dis[...] * t + b[...]
    hrelu = jnp.maximum(pre, 0.0)
    a = g[...] * lax.rsqrt(rv[...] + eps)
    hbn = (hrelu - rm[...]) * a + be[...]
    z = dis[...] * jnp.dot(hbn, w[...], preferred_element_type=jnp.float32)
    h = z.shape[1] // 2
    o0[...] = z[:, :h]
    o1[...] = z[:, h:]


@functools.lru_cache(maxsize=None)
def _tc_mid_kernel(din, dout):
    hin, hout = din // 2, dout // 2
    return pl.pallas_call(
        functools.partial(_tc_mid_body, 1e-5),
        grid=(_GRID,),
        in_specs=[_row_spec(hin), _row_spec(hin), _row_spec(hin),
                  _row_spec(hin), _row_spec(1),
                  _vec_spec(din), _vec_spec(din), _vec_spec(din),
                  _vec_spec(din), _vec_spec(din), _w_spec(din, dout)],
        out_specs=(_row_spec(hout), _row_spec(hout)),
        out_shape=(jax.ShapeDtypeStruct((NP, hout), jnp.float32),
                   jax.ShapeDtypeStruct((NP, hout), jnp.float32)),
    )


def _tc_last_body(ncls, s0, s1, z0, z1, dis, b, o):
    t = jnp.concatenate([s0[...] + z0[...], s1[...] + z1[...]], axis=1)
    t = dis[...] * t + b[...]
    col = lax.broadcasted_iota(jnp.int32, t.shape, 1)
    mask = col < ncls
    neg = jnp.float32(-3.4e38)
    m = jnp.max(jnp.where(mask, t, neg), axis=1, keepdims=True)
    ex = jnp.where(mask, jnp.exp(t - m), 0.0)
    lse = jnp.log(jnp.sum(ex, axis=1, keepdims=True))
    o[...] = t - m - lse


@functools.lru_cache(maxsize=None)
def _tc_last_kernel(din, ncls):
    hin = din // 2
    return pl.pallas_call(
        functools.partial(_tc_last_body, ncls),
        grid=(_GRID,),
        in_specs=[_row_spec(hin), _row_spec(hin), _row_spec(hin),
                  _row_spec(hin), _row_spec(1), _vec_spec(din)],
        out_specs=_row_spec(din),
        out_shape=jax.ShapeDtypeStruct((NP, din), jnp.float32),
    )


def _pad2(a, r, c):
    return jnp.pad(a, ((0, r - a.shape[0]), (0, c - a.shape[1])))


def _padv(a, c, value=0.0):
    return jnp.pad(a, (0, c - a.shape[0]), constant_values=value)[None, :]


def kernel(x, edge_index, W1, b1, W2, b2, W3, b3, W4, b4, W5, b5,
           g1, be1, rm1, rv1, g2, be2, rm2, rv2, g3, be3, rm3, rv3,
           g4, be4, rm4, rv4):
    src = edge_index[0]
    dst = edge_index[1]
    src4 = src.reshape(NSUB, NCH, NBC, EB)
    dst4 = dst.reshape(NSUB, NCH, NBC, EB)

    Ws = [W1, W2, W3, W4, W5]
    bs = [b1, b2, b3, b4, b5]
    gs = [g1, g2, g3, g4]
    bes = [be1, be2, be3, be4]
    rms = [rm1, rm2, rm3, rm4]
    rvs = [rv1, rv2, rv3, rv4]

    xp = _pad2(x, NP, DIMS_P[0])
    wp = [_pad2(Ws[i], DIMS_P[i], DIMS_P[i + 1]) for i in range(5)]
    bp = [_padv(bs[i], DIMS_P[i + 1]) for i in range(5)]
    gp = [_padv(gs[i], DIMS_P[i + 1]) for i in range(4)]
    bep = [_padv(bes[i], DIMS_P[i + 1]) for i in range(4)]
    rmp = [_padv(rms[i], DIMS_P[i + 1]) for i in range(4)]
    rvp = [_padv(rvs[i], DIMS_P[i + 1], 1.0) for i in range(4)]

    zeros1 = jnp.zeros((ROWS_PER_SUB, DW), jnp.float32)
    ones1 = jnp.ones((EB, DW), jnp.float32)

    dstw = dst.reshape(NSUB * NSC, _DEG_NB, EB)
    deg0, deg1 = _sc_degree_kernel()(dstw, ones1, zeros1)
    z0, z1, dis = _tc_first_kernel(DIMS_P[0])(xp, deg0, deg1)

    d2 = DIMS_P[0] // 2
    zer = jnp.zeros((ROWS_PER_SUB, d2), jnp.float32)
    s0, s1 = _sc_agg_kernel(d2)(src4, dst4, z0, z1, zer)
    z0, z1 = _tc_mid1_kernel(DIMS_P[0], DIMS_P[1], DIMS_P[2])(
        s0, s1, z0, z1, dis, wp[0], bp[0], gp[0], bep[0], rmp[0], rvp[0],
        wp[1])

    for l in range(2, 5):
        d2 = DIMS_P[l] // 2
        zer = jnp.zeros((ROWS_PER_SUB, d2), jnp.float32)
        s0, s1 = _sc_agg_kernel(d2)(src4, dst4, z0, z1, zer)
        z0, z1 = _tc_mid_kernel(DIMS_P[l], DIMS_P[l + 1])(
            s0, s1, z0, z1, dis, bp[l - 1], gp[l - 1], bep[l - 1],
            rmp[l - 1], rvp[l - 1], wp[l])

    d2 = DIMS_P[5] // 2
    zer = jnp.zeros((ROWS_PER_SUB, d2), jnp.float32)
    s0, s1 = _sc_agg_kernel(d2)(src4, dst4, z0, z1, zer)
    out = _tc_last_kernel(DIMS_P[5], 17)(s0, s1, z0, z1, dis, bp[4])
    return out[:N, :17]


# speedup vs baseline: 23.9620x; 1.0002x over previous
import functools

import jax
import jax.numpy as jnp
from jax import lax
from jax.experimental import pallas as pl
from jax.experimental.pallas import tpu as pltpu
from jax.experimental.pallas import tpu_sc as plsc

N = 10000
NP = 10240
E = 320000
NSC = 2
NSUB = 16
EB = 80

DIMS_P = [128, 224, 160, 128, 64, 32]
ROWS_PER_SUB = NP // NSUB
DW = 16
NB = (E // NSUB) // EB
NCH = 5
NBC = NB // NCH

_mesh = functools.partial(
    plsc.VectorSubcoreMesh,
    core_axis_name="c", subcore_axis_name="s",
    num_cores=NSC, num_subcores=NSUB,
)

_SC_PARAMS = pltpu.CompilerParams(use_tc_tiling_on_sc=False,
                                  skip_device_barrier=True)


_DEG_NB = (E // (NSC * NSUB)) // EB
_DEG_RING = 8


def _sc_degree_body(dst3, ones, zeros, deg0, deg1, acc, didx, obuf, *ssems):
    s = lax.axis_index("s")
    c = lax.axis_index("c")
    r0 = s * ROWS_PER_SUB
    pltpu.sync_copy(zeros, acc.at[pl.ds(r0, ROWS_PER_SUB)])
    pltpu.sync_copy(ones, obuf)
    w = s * NSC + c
    pltpu.sync_copy(dst3.at[w], didx)
    plsc.subcore_barrier()

    def sissue(i, b):
        pltpu.async_copy(obuf, acc.at[didx.at[i]], ssems[b], add=True)

    def swait(i, b):
        pltpu.make_async_copy(obuf, acc.at[didx.at[i]], ssems[b]).wait()

    for b in range(_DEG_RING):
        sissue(b, b)

    @pl.loop(0, _DEG_NB // _DEG_RING - 1)
    def _segments(j):
        i0 = j * _DEG_RING
        for b in range(_DEG_RING):
            swait(i0 + b, b)
            sissue(i0 + _DEG_RING + b, b)

    i0 = _DEG_NB - _DEG_RING - (_DEG_NB % _DEG_RING)
    for k in range(_DEG_NB % _DEG_RING):
        b = k % _DEG_RING
        swait(i0 + k, b)
        sissue(i0 + _DEG_RING + k, b)
    for k in range(_DEG_RING):
        i = _DEG_NB - _DEG_RING + k
        swait(i, i % _DEG_RING)

    plsc.subcore_barrier()

    @pl.when(c == 0)
    def _():
        pltpu.sync_copy(acc.at[pl.ds(r0, ROWS_PER_SUB)],
                        deg0.at[pl.ds(r0, ROWS_PER_SUB)])

    @pl.when(c == 1)
    def _():
        pltpu.sync_copy(acc.at[pl.ds(r0, ROWS_PER_SUB)],
                        deg1.at[pl.ds(r0, ROWS_PER_SUB)])


@functools.lru_cache(maxsize=None)
def _sc_degree_kernel():
    return pl.kernel(
        _sc_degree_body,
        out_type=(jax.ShapeDtypeStruct((NP, DW), jnp.float32),
                  jax.ShapeDtypeStruct((NP, DW), jnp.float32)),
        mesh=_mesh(),
        compiler_params=_SC_PARAMS,
        scratch_types=[
            pltpu.VMEM_SHARED((NP, DW), jnp.float32),
            pltpu.VMEM((_DEG_NB, EB), jnp.int32),
            pltpu.VMEM((EB, DW), jnp.float32),
        ] + [pltpu.SemaphoreType.DMA for _ in range(_DEG_RING)],
    )


def _sc_agg_body(nbuf, src4, dst4, z0, z1, zeros, out0, out1,
                 acc, sidx, didx, *bufs_and_sems):
    s = lax.axis_index("s")
    c = lax.axis_index("c")
    r0 = s * ROWS_PER_SUB
    ebufs = bufs_and_sems[:nbuf]
    gsems = bufs_and_sems[nbuf:2 * nbuf]
    ssems = bufs_and_sems[2 * nbuf:]

    pltpu.sync_copy(zeros, acc.at[pl.ds(r0, ROWS_PER_SUB)])

    def gissue(i, b):
        @pl.when(c == 0)
        def _():
            pltpu.async_copy(z0.at[sidx.at[i]], ebufs[b], gsems[b])

        @pl.when(c == 1)
        def _():
            pltpu.async_copy(z1.at[sidx.at[i]], ebufs[b], gsems[b])

    def gwait(i, b):
        @pl.when(c == 0)
        def _():
            pltpu.make_async_copy(z0.at[sidx.at[i]], ebufs[b], gsems[b]).wait()

        @pl.when(c == 1)
        def _():
            pltpu.make_async_copy(z1.at[sidx.at[i]], ebufs[b], gsems[b]).wait()

    def sissue(i, b):
        pltpu.async_copy(ebufs[b], acc.at[didx.at[i]], ssems[b], add=True)

    def swait(i, b):
        pltpu.make_async_copy(ebufs[b], acc.at[didx.at[i]], ssems[b]).wait()

    plsc.subcore_barrier()

    lag = 2

    @pl.loop(0, NCH)
    def _chunk(ch):
        pltpu.sync_copy(src4.at[s, ch], sidx)
        pltpu.sync_copy(dst4.at[s, ch], didx)
        for b in range(nbuf - lag):
            gissue(b, b)
        for i in range(nbuf):
            gwait(i, i)
            sissue(i, i)
            if i < lag:
                gissue(i + nbuf - lag, i + nbuf - lag)
            else:
                swait(i - lag, (i - lag) % nbuf)
                gissue(i + nbuf - lag, (i + nbuf - lag) % nbuf)

        @pl.loop(1, NBC // nbuf)
        def _segments(j):
            i0 = j * nbuf
            for b in range(nbuf):
                i = i0 + b
                gwait(i, b)
                sissue(i, b)
                swait(i - lag, (b - lag) % nbuf)

                @pl.when(i + nbuf - lag < NBC)
                def _(i=i, b=b):
                    gissue(i + nbuf - lag, (b + nbuf - lag) % nbuf)

        for k in range(lag):
            swait(NBC - lag + k, (NBC - lag + k) % nbuf)

    plsc.subcore_barrier()

    @pl.when(c == 0)
    def _():
        pltpu.sync_copy(acc.at[pl.ds(r0, ROWS_PER_SUB)],
                        out0.at[pl.ds(r0, ROWS_PER_SUB)])

    @pl.when(c == 1)
    def _():
        pltpu.sync_copy(acc.at[pl.ds(r0, ROWS_PER_SUB)],
                        out1.at[pl.ds(r0, ROWS_PER_SUB)])


@functools.lru_cache(maxsize=None)
def _sc_agg_kernel(d2):
    nbuf = {112: 5, 80: 10, 64: 10, 32: 10, 16: 10}[d2]
    return pl.kernel(
        functools.partial(_sc_agg_body, nbuf),
        out_type=(jax.ShapeDtypeStruct((NP, d2), jnp.float32),
                  jax.ShapeDtypeStruct((NP, d2), jnp.float32)),
        mesh=_mesh(),
        compiler_params=_SC_PARAMS,
        scratch_types=(
            [pltpu.VMEM_SHARED((NP, d2), jnp.float32),
             pltpu.VMEM((NBC, EB), jnp.int32),
             pltpu.VMEM((NBC, EB), jnp.int32)]
            + [pltpu.VMEM((EB, d2), jnp.float32) for _ in range(nbuf)]
            + [pltpu.SemaphoreType.DMA for _ in range(2 * nbuf)]
        ),
    )


_BLK = 256
_GRID = NP // _BLK


def _row_spec(d):
    return pl.BlockSpec((_BLK, d), lambda i: (i, 0))


def _vec_spec(d):
    return pl.BlockSpec((1, d), lambda i: (0, 0))


def _w_spec(din, dout):
    return pl.BlockSpec((din, dout), lambda i: (0, 0))


def _tc_first_body(x, deg0, deg1, z0, z1, dis):
    d = deg0[...][:, :1] + deg1[...][:, :1]
    di = lax.rsqrt(jnp.maximum(d, 1.0))
    z = di * x[...]
    h = z.shape[1] // 2
    z0[...] = z[:, :h]
    z1[...] = z[:, h:]
    dis[...] = di


@functools.lru_cache(maxsize=None)
def _tc_first_kernel(din):
    h = din // 2
    return pl.pallas_call(
        _tc_first_body,
        grid=(_GRID,),
        in_specs=[_row_spec(din), _row_spec(DW), _row_spec(DW)],
        out_specs=(_row_spec(h), _row_spec(h), _row_spec(1)),
        out_shape=(jax.ShapeDtypeStruct((NP, h), jnp.float32),
                   jax.ShapeDtypeStruct((NP, h), jnp.float32),
                   jax.ShapeDtypeStruct((NP, 1), jnp.float32)),
    )


def _tc_mid1_body(eps, s0, s1, z0, z1, dis, w1, b, g, be, rm, rv, w2, o0, o1):
    t = jnp.concatenate([s0[...] + z0[...], s1[...] + z1[...]], axis=1)
    u = jnp.dot(dis[...] * t, w1[...],
                preferred_element_type=jnp.float32) + b[...]
    hrelu = jnp.maximum(u, 0.0)
    a = g[...] * lax.rsqrt(rv[...] + eps)
    hbn = (hrelu - rm[...]) * a + be[...]
    z = dis[...] * jnp.dot(hbn, w2[...], preferred_element_type=jnp.float32)
    h = z.shape[1] // 2
    o0[...] = z[:, :h]
    o1[...] = z[:, h:]


@functools.lru_cache(maxsize=None)
def _tc_mid1_kernel(din, dmid, dout):
    hin, hout = din // 2, dout // 2
    return pl.pallas_call(
        functools.partial(_tc_mid1_body, 1e-5),
        grid=(_GRID,),
        in_specs=[_row_spec(hin), _row_spec(hin), _row_spec(hin),
                  _row_spec(hin), _row_spec(1), _w_spec(din, dmid),
                  _vec_spec(dmid), _vec_spec(dmid), _vec_spec(dmid),
                  _vec_spec(dmid), _vec_spec(dmid), _w_spec(dmid, dout)],
        out_specs=(_row_spec(hout), _row_spec(hout)),
        out_shape=(jax.ShapeDtypeStruct((NP, hout), jnp.float32),
                   jax.ShapeDtypeStruct((NP, hout), jnp.float32)),
    )


def _tc_mid_body(eps, s0, s1, z0, z1, dis, b, g, be, rm, rv, w, o0, o1):
    t = jnp.concatenate([s0[...] + z0[...], s1[...] + z1[...]], axis=1)
    pre = dis[...] * t + b[...]
    hrelu = jnp.maximum(pre, 0.0)
    a = g[...] * lax.rsqrt(rv[...] + eps)
    hbn = (hrelu - rm[...]) * a + be[...]
    z = dis[...] * jnp.dot(hbn, w[...], preferred_element_type=jnp.float32)
    h = z.shape[1] // 2
    o0[...] = z[:, :h]
    o1[...] = z[:, h:]


@functools.lru_cache(maxsize=None)
def _tc_mid_kernel(din, dout):
    hin, hout = din // 2, dout // 2
    return pl.pallas_call(
        functools.partial(_tc_mid_body, 1e-5),
        grid=(_GRID,),
        in_specs=[_row_spec(hin), _row_spec(hin), _row_spec(hin),
                  _row_spec(hin), _row_spec(1),
                  _vec_spec(din), _vec_spec(din), _vec_spec(din),
                  _vec_spec(din), _vec_spec(din), _w_spec(din, dout)],
        out_specs=(_row_spec(hout), _row_spec(hout)),
        out_shape=(jax.ShapeDtypeStruct((NP, hout), jnp.float32),
                   jax.ShapeDtypeStruct((NP, hout), jnp.float32)),
    )


def _tc_last_body(ncls, s0, s1, z0, z1, dis, b, o):
    t = jnp.concatenate([s0[...] + z0[...], s1[...] + z1[...]], axis=1)
    t = dis[...] * t + b[...]
    col = lax.broadcasted_iota(jnp.int32, t.shape, 1)
    mask = col < ncls
    neg = jnp.float32(-3.4e38)
    m = jnp.max(jnp.where(mask, t, neg), axis=1, keepdims=True)
    ex = jnp.where(mask, jnp.exp(t - m), 0.0)
    lse = jnp.log(jnp.sum(ex, axis=1, keepdims=True))
    o[...] = t - m - lse


@functools.lru_cache(maxsize=None)
def _tc_last_kernel(din, ncls):
    hin = din // 2
    return pl.pallas_call(
        functools.partial(_tc_last_body, ncls),
        grid=(_GRID,),
        in_specs=[_row_spec(hin), _row_spec(hin), _row_spec(hin),
                  _row_spec(hin), _row_spec(1), _vec_spec(din)],
        out_specs=_row_spec(din),
        out_shape=jax.ShapeDtypeStruct((NP, din), jnp.float32),
    )


def _pad2(a, r, c):
    return jnp.pad(a, ((0, r - a.shape[0]), (0, c - a.shape[1])))


def _padv(a, c, value=0.0):
    return jnp.pad(a, (0, c - a.shape[0]), constant_values=value)[None, :]


def kernel(x, edge_index, W1, b1, W2, b2, W3, b3, W4, b4, W5, b5,
           g1, be1, rm1, rv1, g2, be2, rm2, rv2, g3, be3, rm3, rv3,
           g4, be4, rm4, rv4):
    src = edge_index[0]
    dst = edge_index[1]
    src4 = src.reshape(NSUB, NCH, NBC, EB)
    dst4 = dst.reshape(NSUB, NCH, NBC, EB)

    Ws = [W1, W2, W3, W4, W5]
    bs = [b1, b2, b3, b4, b5]
    gs = [g1, g2, g3, g4]
    bes = [be1, be2, be3, be4]
    rms = [rm1, rm2, rm3, rm4]
    rvs = [rv1, rv2, rv3, rv4]

    xp = _pad2(x, NP, DIMS_P[0])
    wp = [_pad2(Ws[i], DIMS_P[i], DIMS_P[i + 1]) for i in range(5)]
    bp = [_padv(bs[i], DIMS_P[i + 1]) for i in range(5)]
    gp = [_padv(gs[i], DIMS_P[i + 1]) for i in range(4)]
    bep = [_padv(bes[i], DIMS_P[i + 1]) for i in range(4)]
    rmp = [_padv(rms[i], DIMS_P[i + 1]) for i in range(4)]
    rvp = [_padv(rvs[i], DIMS_P[i + 1], 1.0) for i in range(4)]

    zeros1 = jnp.zeros((ROWS_PER_SUB, DW), jnp.float32)
    ones1 = jnp.ones((EB, DW), jnp.float32)

    dstw = dst.reshape(NSUB * NSC, _DEG_NB, EB)
    deg0, deg1 = _sc_degree_kernel()(dstw, ones1, zeros1)
    z0, z1, dis = _tc_first_kernel(DIMS_P[0])(xp, deg0, deg1)

    d2 = DIMS_P[0] // 2
    zer = jnp.zeros((ROWS_PER_SUB, d2), jnp.float32)
    s0, s1 = _sc_agg_kernel(d2)(src4, dst4, z0, z1, zer)
    z0, z1 = _tc_mid1_kernel(DIMS_P[0], DIMS_P[1], DIMS_P[2])(
        s0, s1, z0, z1, dis, wp[0], bp[0], gp[0], bep[0], rmp[0], rvp[0],
        wp[1])

    for l in range(2, 5):
        d2 = DIMS_P[l] // 2
        zer = jnp.zeros((ROWS_PER_SUB, d2), jnp.float32)
        s0, s1 = _sc_agg_kernel(d2)(src4, dst4, z0, z1, zer)
        z0, z1 = _tc_mid_kernel(DIMS_P[l], DIMS_P[l + 1])(
            s0, s1, z0, z1, dis, bp[l - 1], gp[l - 1], bep[l - 1],
            rmp[l - 1], rvp[l - 1], wp[l])

    d2 = DIMS_P[5] // 2
    zer = jnp.zeros((ROWS_PER_SUB, d2), jnp.float32)
    s0, s1 = _sc_agg_kernel(d2)(src4, dst4, z0, z1, zer)
    out = _tc_last_kernel(DIMS_P[5], 17)(s0, s1, z0, z1, dis, bp[4])
    return out[:N, :17]


# speedup vs baseline: 25.1560x; 1.0498x over previous
import functools

import jax
import jax.numpy as jnp
from jax import lax
from jax.experimental import pallas as pl
from jax.experimental.pallas import tpu as pltpu
from jax.experimental.pallas import tpu_sc as plsc

N = 10000
NP = 10240
E = 320000
NSC = 2
NSUB = 16
EB = 80

DIMS_P = [128, 224, 160, 128, 64, 32]
ROWS_PER_SUB = NP // NSUB
DW = 16
NB = (E // NSUB) // EB
NCH = 5
NBC = NB // NCH

_mesh = functools.partial(
    plsc.VectorSubcoreMesh,
    core_axis_name="c", subcore_axis_name="s",
    num_cores=NSC, num_subcores=NSUB,
)

_SC_PARAMS = pltpu.CompilerParams(use_tc_tiling_on_sc=False)


_DEG_NB = (E // (NSC * NSUB)) // EB
_DEG_RING = 8


def _sc_degree_body(dst3, ones, zeros, deg0, deg1, acc, didx, obuf, *ssems):
    s = lax.axis_index("s")
    c = lax.axis_index("c")
    r0 = s * ROWS_PER_SUB
    pltpu.sync_copy(zeros, acc.at[pl.ds(r0, ROWS_PER_SUB)])
    pltpu.sync_copy(ones, obuf)
    w = s * NSC + c
    pltpu.sync_copy(dst3.at[w], didx)
    plsc.subcore_barrier()

    def sissue(i, b):
        pltpu.async_copy(obuf, acc.at[didx.at[i]], ssems[b], add=True)

    def swait(i, b):
        pltpu.make_async_copy(obuf, acc.at[didx.at[i]], ssems[b]).wait()

    for b in range(_DEG_RING):
        sissue(b, b)

    @pl.loop(0, _DEG_NB // _DEG_RING - 1)
    def _segments(j):
        i0 = j * _DEG_RING
        for b in range(_DEG_RING):
            swait(i0 + b, b)
            sissue(i0 + _DEG_RING + b, b)

    i0 = _DEG_NB - _DEG_RING - (_DEG_NB % _DEG_RING)
    for k in range(_DEG_NB % _DEG_RING):
        b = k % _DEG_RING
        swait(i0 + k, b)
        sissue(i0 + _DEG_RING + k, b)
    for k in range(_DEG_RING):
        i = _DEG_NB - _DEG_RING + k
        swait(i, i % _DEG_RING)

    plsc.subcore_barrier()

    @pl.when(c == 0)
    def _():
        pltpu.sync_copy(acc.at[pl.ds(r0, ROWS_PER_SUB)],
                        deg0.at[pl.ds(r0, ROWS_PER_SUB)])

    @pl.when(c == 1)
    def _():
        pltpu.sync_copy(acc.at[pl.ds(r0, ROWS_PER_SUB)],
                        deg1.at[pl.ds(r0, ROWS_PER_SUB)])


@functools.lru_cache(maxsize=None)
def _sc_degree_kernel():
    return pl.kernel(
        _sc_degree_body,
        out_type=(jax.ShapeDtypeStruct((NP, DW), jnp.float32),
                  jax.ShapeDtypeStruct((NP, DW), jnp.float32)),
        mesh=_mesh(),
        compiler_params=_SC_PARAMS,
        scratch_types=[
            pltpu.VMEM_SHARED((NP, DW), jnp.float32),
            pltpu.VMEM((_DEG_NB, EB), jnp.int32),
            pltpu.VMEM((EB, DW), jnp.float32),
        ] + [pltpu.SemaphoreType.DMA for _ in range(_DEG_RING)],
    )


def _sc_agg_body(nbuf, src4, dst4, z0, z1, zeros, out0, out1,
                 acc, sidx, didx, *bufs_and_sems):
    s = lax.axis_index("s")
    c = lax.axis_index("c")
    r0 = s * ROWS_PER_SUB
    ebufs = bufs_and_sems[:nbuf]
    gsems = bufs_and_sems[nbuf:2 * nbuf]
    ssems = bufs_and_sems[2 * nbuf:]

    pltpu.sync_copy(zeros, acc.at[pl.ds(r0, ROWS_PER_SUB)])

    def gissue(i, b):
        @pl.when(c == 0)
        def _():
            pltpu.async_copy(z0.at[sidx.at[i]], ebufs[b], gsems[b])

        @pl.when(c == 1)
        def _():
            pltpu.async_copy(z1.at[sidx.at[i]], ebufs[b], gsems[b])

    def gwait(i, b):
        @pl.when(c == 0)
        def _():
            pltpu.make_async_copy(z0.at[sidx.at[i]], ebufs[b], gsems[b]).wait()

        @pl.when(c == 1)
        def _():
            pltpu.make_async_copy(z1.at[sidx.at[i]], ebufs[b], gsems[b]).wait()

    def sissue(i, b):
        pltpu.async_copy(ebufs[b], acc.at[didx.at[i]], ssems[b], add=True)

    def swait(i, b):
        pltpu.make_async_copy(ebufs[b], acc.at[didx.at[i]], ssems[b]).wait()

    plsc.subcore_barrier()

    lag = 2

    @pl.loop(0, NCH)
    def _chunk(ch):
        pltpu.sync_copy(src4.at[s, ch], sidx)
        pltpu.sync_copy(dst4.at[s, ch], didx)
        for b in range(nbuf - lag):
            gissue(b, b)
        for i in range(nbuf):
            gwait(i, i)
            sissue(i, i)
            if i < lag:
                gissue(i + nbuf - lag, i + nbuf - lag)
            else:
                swait(i - lag, (i - lag) % nbuf)
                gissue(i + nbuf - lag, (i + nbuf - lag) % nbuf)

        @pl.loop(1, NBC // nbuf)
        def _segments(j):
            i0 = j * nbuf
            for b in range(nbuf):
                i = i0 + b
                gwait(i, b)
                sissue(i, b)
                swait(i - lag, (b - lag) % nbuf)

                @pl.when(i + nbuf - lag < NBC)
                def _(i=i, b=b):
                    gissue(i + nbuf - lag, (b + nbuf - lag) % nbuf)

        for k in range(lag):
            swait(NBC - lag + k, (NBC - lag + k) % nbuf)

    plsc.subcore_barrier()

    d2 = acc.shape[1]

    @pl.when(c == 0)
    def _():
        pltpu.sync_copy(acc.at[pl.ds(r0, ROWS_PER_SUB)],
                        out0.at[pl.ds(r0, ROWS_PER_SUB), pl.ds(0, d2)])

    @pl.when(c == 1)
    def _():
        pltpu.sync_copy(acc.at[pl.ds(r0, ROWS_PER_SUB)],
                        out1.at[pl.ds(r0, ROWS_PER_SUB), pl.ds(0, d2)])


@functools.lru_cache(maxsize=None)
def _sc_agg_kernel(d2):
    nbuf = {112: 5, 80: 10, 64: 10, 32: 10, 16: 10}[d2]
    return pl.kernel(
        functools.partial(_sc_agg_body, nbuf),
        out_type=(jax.ShapeDtypeStruct((NP, 128), jnp.float32),
                  jax.ShapeDtypeStruct((NP, 128), jnp.float32)),
        mesh=_mesh(),
        compiler_params=_SC_PARAMS,
        scratch_types=(
            [pltpu.VMEM_SHARED((NP, d2), jnp.float32),
             pltpu.VMEM((NBC, EB), jnp.int32),
             pltpu.VMEM((NBC, EB), jnp.int32)]
            + [pltpu.VMEM((EB, d2), jnp.float32) for _ in range(nbuf)]
            + [pltpu.SemaphoreType.DMA for _ in range(2 * nbuf)]
        ),
    )


_BLK = 256
_GRID = NP // _BLK


def _row_spec(d):
    return pl.BlockSpec((_BLK, d), lambda i: (i, 0))


def _vec_spec(d):
    return pl.BlockSpec((1, d), lambda i: (0, 0))


def _w_spec(din, dout):
    return pl.BlockSpec((din, dout), lambda i: (0, 0))


def _tc_first_body(x, deg0, deg1, z0, z1, dis):
    d = deg0[...][:, :1] + deg1[...][:, :1]
    di = lax.rsqrt(jnp.maximum(d, 1.0))
    z = di * x[...]
    h = z.shape[1] // 2
    z0[...] = z[:, :h]
    z1[...] = z[:, h:]
    dis[...] = di


@functools.lru_cache(maxsize=None)
def _tc_first_kernel(din):
    h = din // 2
    return pl.pallas_call(
        _tc_first_body,
        grid=(_GRID,),
        in_specs=[_row_spec(din), _row_spec(DW), _row_spec(DW)],
        out_specs=(_row_spec(h), _row_spec(h), _row_spec(1)),
        out_shape=(jax.ShapeDtypeStruct((NP, h), jnp.float32),
                   jax.ShapeDtypeStruct((NP, h), jnp.float32),
                   jax.ShapeDtypeStruct((NP, 1), jnp.float32)),
    )


def _tc_mid1_body(eps, s0, s1, z0, z1, dis, w1, b, g, be, rm, rv, w2, o0, o1):
    hin = z0.shape[1]
    t = jnp.concatenate([s0[...][:, :hin] + z0[...],
                         s1[...][:, :hin] + z1[...]], axis=1)
    u = jnp.dot(dis[...] * t, w1[...],
                preferred_element_type=jnp.float32) + b[...]
    hrelu = jnp.maximum(u, 0.0)
    a = g[...] * lax.rsqrt(rv[...] + eps)
    hbn = (hrelu - rm[...]) * a + be[...]
    z = dis[...] * jnp.dot(hbn, w2[...], preferred_element_type=jnp.float32)
    h = z.shape[1] // 2
    o0[...] = z[:, :h]
    o1[...] = z[:, h:]


@functools.lru_cache(maxsize=None)
def _tc_mid1_kernel(din, dmid, dout):
    hin, hout = din // 2, dout // 2
    return pl.pallas_call(
        functools.partial(_tc_mid1_body, 1e-5),
        grid=(_GRID,),
        in_specs=[_row_spec(128), _row_spec(128), _row_spec(hin),
                  _row_spec(hin), _row_spec(1), _w_spec(din, dmid),
                  _vec_spec(dmid), _vec_spec(dmid), _vec_spec(dmid),
                  _vec_spec(dmid), _vec_spec(dmid), _w_spec(dmid, dout)],
        out_specs=(_row_spec(hout), _row_spec(hout)),
        out_shape=(jax.ShapeDtypeStruct((NP, hout), jnp.float32),
                   jax.ShapeDtypeStruct((NP, hout), jnp.float32)),
    )


def _tc_mid_body(eps, s0, s1, z0, z1, dis, b, g, be, rm, rv, w, o0, o1):
    hin = z0.shape[1]
    t = jnp.concatenate([s0[...][:, :hin] + z0[...],
                         s1[...][:, :hin] + z1[...]], axis=1)
    pre = dis[...] * t + b[...]
    hrelu = jnp.maximum(pre, 0.0)
    a = g[...] * lax.rsqrt(rv[...] + eps)
    hbn = (hrelu - rm[...]) * a + be[...]
    z = dis[...] * jnp.dot(hbn, w[...], preferred_element_type=jnp.float32)
    h = z.shape[1] // 2
    o0[...] = z[:, :h]
    o1[...] = z[:, h:]


@functools.lru_cache(maxsize=None)
def _tc_mid_kernel(din, dout):
    hin, hout = din // 2, dout // 2
    return pl.pallas_call(
        functools.partial(_tc_mid_body, 1e-5),
        grid=(_GRID,),
        in_specs=[_row_spec(128), _row_spec(128), _row_spec(hin),
                  _row_spec(hin), _row_spec(1),
                  _vec_spec(din), _vec_spec(din), _vec_spec(din),
                  _vec_spec(din), _vec_spec(din), _w_spec(din, dout)],
        out_specs=(_row_spec(hout), _row_spec(hout)),
        out_shape=(jax.ShapeDtypeStruct((NP, hout), jnp.float32),
                   jax.ShapeDtypeStruct((NP, hout), jnp.float32)),
    )


def _tc_last_body(ncls, s0, s1, z0, z1, dis, b, o):
    hin = z0.shape[1]
    t = jnp.concatenate([s0[...][:, :hin] + z0[...],
                         s1[...][:, :hin] + z1[...]], axis=1)
    t = dis[...] * t + b[...]
    col = lax.broadcasted_iota(jnp.int32, t.shape, 1)
    mask = col < ncls
    neg = jnp.float32(-3.4e38)
    m = jnp.max(jnp.where(mask, t, neg), axis=1, keepdims=True)
    ex = jnp.where(mask, jnp.exp(t - m), 0.0)
    lse = jnp.log(jnp.sum(ex, axis=1, keepdims=True))
    o[...] = t - m - lse


@functools.lru_cache(maxsize=None)
def _tc_last_kernel(din, ncls):
    hin = din // 2
    return pl.pallas_call(
        functools.partial(_tc_last_body, ncls),
        grid=(_GRID,),
        in_specs=[_row_spec(128), _row_spec(128), _row_spec(hin),
                  _row_spec(hin), _row_spec(1), _vec_spec(din)],
        out_specs=_row_spec(din),
        out_shape=jax.ShapeDtypeStruct((NP, din), jnp.float32),
    )


def _pad2(a, r, c):
    return jnp.pad(a, ((0, r - a.shape[0]), (0, c - a.shape[1])))


def _padv(a, c, value=0.0):
    return jnp.pad(a, (0, c - a.shape[0]), constant_values=value)[None, :]


def kernel(x, edge_index, W1, b1, W2, b2, W3, b3, W4, b4, W5, b5,
           g1, be1, rm1, rv1, g2, be2, rm2, rv2, g3, be3, rm3, rv3,
           g4, be4, rm4, rv4):
    src = edge_index[0]
    dst = edge_index[1]
    src4 = src.reshape(NSUB, NCH, NBC, EB)
    dst4 = dst.reshape(NSUB, NCH, NBC, EB)

    Ws = [W1, W2, W3, W4, W5]
    bs = [b1, b2, b3, b4, b5]
    gs = [g1, g2, g3, g4]
    bes = [be1, be2, be3, be4]
    rms = [rm1, rm2, rm3, rm4]
    rvs = [rv1, rv2, rv3, rv4]

    xp = _pad2(x, NP, DIMS_P[0])
    wp = [_pad2(Ws[i], DIMS_P[i], DIMS_P[i + 1]) for i in range(5)]
    bp = [_padv(bs[i], DIMS_P[i + 1]) for i in range(5)]
    gp = [_padv(gs[i], DIMS_P[i + 1]) for i in range(4)]
    bep = [_padv(bes[i], DIMS_P[i + 1]) for i in range(4)]
    rmp = [_padv(rms[i], DIMS_P[i + 1]) for i in range(4)]
    rvp = [_padv(rvs[i], DIMS_P[i + 1], 1.0) for i in range(4)]

    zeros1 = jnp.zeros((ROWS_PER_SUB, DW), jnp.float32)
    ones1 = jnp.ones((EB, DW), jnp.float32)

    dstw = dst.reshape(NSUB * NSC, _DEG_NB, EB)
    deg0, deg1 = _sc_degree_kernel()(dstw, ones1, zeros1)
    z0, z1, dis = _tc_first_kernel(DIMS_P[0])(xp, deg0, deg1)

    d2 = DIMS_P[0] // 2
    zer = jnp.zeros((ROWS_PER_SUB, d2), jnp.float32)
    s0, s1 = _sc_agg_kernel(d2)(src4, dst4, z0, z1, zer)
    z0, z1 = _tc_mid1_kernel(DIMS_P[0], DIMS_P[1], DIMS_P[2])(
        s0, s1, z0, z1, dis, wp[0], bp[0], gp[0], bep[0], rmp[0], rvp[0],
        wp[1])

    for l in range(2, 5):
        d2 = DIMS_P[l] // 2
        zer = jnp.zeros((ROWS_PER_SUB, d2), jnp.float32)
        s0, s1 = _sc_agg_kernel(d2)(src4, dst4, z0, z1, zer)
        z0, z1 = _tc_mid_kernel(DIMS_P[l], DIMS_P[l + 1])(
            s0, s1, z0, z1, dis, bp[l - 1], gp[l - 1], bep[l - 1],
            rmp[l - 1], rvp[l - 1], wp[l])

    d2 = DIMS_P[5] // 2
    zer = jnp.zeros((ROWS_PER_SUB, d2), jnp.float32)
    s0, s1 = _sc_agg_kernel(d2)(src4, dst4, z0, z1, zer)
    out = _tc_last_kernel(DIMS_P[5], 17)(s0, s1, z0, z1, dis, bp[4])
    return out[:N, :17]
